# baseline (speedup 1.0000x reference)
import functools

import jax
import jax.numpy as jnp
from jax.experimental import pallas as pl

HEADS = 10
D = 20
N_GRAPHS = 64


def _linear_block(x_ref, w_ref, b_ref, o_ref):
    o_ref[...] = (
        jnp.dot(x_ref[...], w_ref[...], preferred_element_type=jnp.float32)
        + b_ref[...]
    )


def _linear(x, W, b, block_rows=2000):
    n, k = x.shape
    m = W.shape[0]
    Wt = W.T
    b2 = b.reshape(1, m)
    grid = (n // block_rows,)
    return pl.pallas_call(
        _linear_block,
        grid=grid,
        in_specs=[
            pl.BlockSpec((block_rows, k), lambda i: (i, 0)),
            pl.BlockSpec((k, m), lambda i: (0, 0)),
            pl.BlockSpec((1, m), lambda i: (0, 0)),
        ],
        out_specs=pl.BlockSpec((block_rows, m), lambda i: (i, 0)),
        out_shape=jax.ShapeDtypeStruct((n, m), jnp.float32),
    )(x, Wt, b2)


def _gatv2(x, edge_index, edge_attr, p):
    n = x.shape[0]
    loop = jnp.arange(n, dtype=edge_index.dtype)
    ei = jnp.concatenate([edge_index, jnp.stack([loop, loop])], axis=1)
    ea_mean = jnp.mean(edge_attr, axis=0, keepdims=True)
    ea = jnp.concatenate(
        [edge_attr, jnp.broadcast_to(ea_mean, (n, edge_attr.shape[1]))], axis=0
    )
    src, dst = ei[0], ei[1]
    x_l = _linear(x, p['gat_Wl'], p['gat_bl']).reshape(n, HEADS, D)
    x_r = _linear(x, p['gat_Wr'], p['gat_br']).reshape(n, HEADS, D)
    e = _linear(ea, p['gat_We'], jnp.zeros((HEADS * D,), jnp.float32),
                block_rows=2000 if ea.shape[0] % 2000 == 0 else 1000)
    e = e.reshape(-1, HEADS, D)
    xj = x_l[src]
    m = jax.nn.leaky_relu(xj + x_r[dst] + e, 0.2)
    alpha = jnp.sum(m * p['gat_att'][None], axis=-1)
    amax = jax.ops.segment_max(alpha, dst, num_segments=n)
    ex = jnp.exp(alpha - amax[dst])
    den = jax.ops.segment_sum(ex, dst, num_segments=n)
    a = ex / (den[dst] + 1e-16)
    agg = jax.ops.segment_sum(xj * a[:, :, None], dst, num_segments=n)
    return agg.mean(axis=1) + p['gat_bias']


def _ggc(x, edge_index, p):
    src, dst = edge_index[0], edge_index[1]
    m = x @ p['ggc_weight']
    agg = jax.ops.segment_sum(m[src], dst, num_segments=x.shape[0])
    gi = agg @ p['gru_Wih'].T + p['gru_bih']
    gh = x @ p['gru_Whh'].T + p['gru_bhh']
    ir, iz, inn = jnp.split(gi, 3, axis=1)
    hr, hz, hn = jnp.split(gh, 3, axis=1)
    r = jax.nn.sigmoid(ir + hr)
    z = jax.nn.sigmoid(iz + hz)
    nt = jnp.tanh(inn + r * hn)
    return (1.0 - z) * nt + z * x


def kernel(x, edge_index, edge_attr, batch, params):
    h0 = x
    m1 = _gatv2(x, edge_index, edge_attr, params)
    h1 = _ggc(m1, edge_index, params)
    m2 = _gatv2(h1, edge_index, edge_attr, params)
    h2 = _ggc(m2, edge_index, params)
    H = jnp.concatenate([h0, h1, h2], axis=1)
    src, dst = edge_index[0], edge_index[1]
    agg = jax.ops.segment_sum(H[src], dst, num_segments=H.shape[0])
    score = (agg @ params['pool_Wrel'].T + params['pool_brel']
             + H @ params['pool_Wroot'].T).reshape(-1)
    counts = jnp.bincount(batch, length=N_GRAPHS)
    k = jnp.ceil(0.3 * counts.astype(jnp.float32)).astype(jnp.int32)
    order = jnp.lexsort((-score, batch))
    starts = jnp.cumsum(counts) - counts
    pos = jnp.arange(batch.shape[0], dtype=jnp.int32) - starts[batch[order]].astype(jnp.int32)
    rank = jnp.zeros(batch.shape[0], dtype=jnp.int32).at[order].set(pos)
    mask = (rank < k[batch]).astype(H.dtype)
    Hs = H * jnp.tanh(score)[:, None] * mask[:, None]
    g = jax.ops.segment_sum(Hs, batch, num_segments=N_GRAPHS)
    g = jax.nn.leaky_relu(g @ params['fc1_W'].T + params['fc1_b'], 0.01)
    g = jax.nn.leaky_relu(g @ params['fc2_W'].T + params['fc2_b'], 0.01)
    out = g @ params['out_W'].T + params['out_b']
    return out.reshape(-1)


# speedup vs baseline: 13.2971x; 13.2971x over previous
import functools

import jax
import jax.numpy as jnp
from jax import lax
from jax.experimental import pallas as pl
from jax.experimental.pallas import tpu as pltpu
from jax.experimental.pallas import tpu_sc as plsc

HEADS = 10
D = 20
N_GRAPHS = 64
N = 10000
E = 160000
NPAD = 10016
NW = 32
BE = 128
EPAD = 163840
KB = EPAD // (NW * BE)
WF = 208
NP2 = 10240

_SC_PARAMS = pltpu.CompilerParams(use_tc_tiling_on_sc=False)


def _linear_block(x_ref, w_ref, b_ref, o_ref):
    o_ref[...] = (
        jnp.dot(x_ref[...], w_ref[...], preferred_element_type=jnp.float32)
        + b_ref[...]
    )


def _linear(x, Wt, b, block_rows):
    n, k = x.shape
    m = Wt.shape[1]
    b2 = b.reshape(1, m)
    return pl.pallas_call(
        _linear_block,
        grid=(n // block_rows,),
        in_specs=[
            pl.BlockSpec((block_rows, k), lambda i: (i, 0)),
            pl.BlockSpec((k, m), lambda i: (0, 0)),
            pl.BlockSpec((1, m), lambda i: (0, 0)),
        ],
        out_specs=pl.BlockSpec((block_rows, m), lambda i: (i, 0)),
        out_shape=jax.ShapeDtypeStruct((n, m), jnp.float32),
    )(x, Wt, b2)


def _sc_mesh():
    return plsc.VectorSubcoreMesh(core_axis_name="c", subcore_axis_name="s")


def _sc_gather2(tblA, tblB, src3, dst3):
    WA = tblA.shape[1]
    WB = tblB.shape[1]

    @functools.partial(
        pl.kernel,
        mesh=_sc_mesh(),
        out_type=[
            jax.ShapeDtypeStruct((EPAD, WA), jnp.float32),
            jax.ShapeDtypeStruct((EPAD, WB), jnp.float32),
        ],
        compiler_params=_SC_PARAMS,
        scratch_types=[
            pltpu.VMEM((KB, BE), jnp.int32),
            pltpu.VMEM((KB, BE), jnp.int32),
            pltpu.VMEM((BE, WA), jnp.float32),
            pltpu.VMEM((BE, WB), jnp.float32),
        ],
    )
    def k(tA_h, tB_h, src_h, dst_h, oA_h, oB_h, sidx, didx, bufA, bufB):
        cid = lax.axis_index("c")
        sid = lax.axis_index("s")
        wid = sid * 2 + cid
        pltpu.sync_copy(src_h.at[wid], sidx)
        pltpu.sync_copy(dst_h.at[wid], didx)

        @pl.loop(0, KB)
        def _(j):
            base = wid * (KB * BE) + j * BE
            pltpu.sync_copy(tA_h.at[sidx.at[j]], bufA)
            pltpu.sync_copy(bufA, oA_h.at[pl.ds(base, BE)])
            pltpu.sync_copy(tB_h.at[didx.at[j]], bufB)
            pltpu.sync_copy(bufB, oB_h.at[pl.ds(base, BE)])

    return k(tblA, tblB, src3, dst3)


def _sc_gather1(tbl, idx3):
    W = tbl.shape[1]

    @functools.partial(
        pl.kernel,
        mesh=_sc_mesh(),
        out_type=jax.ShapeDtypeStruct((EPAD, W), jnp.float32),
        compiler_params=_SC_PARAMS,
        scratch_types=[
            pltpu.VMEM((KB, BE), jnp.int32),
            pltpu.VMEM((BE, W), jnp.float32),
        ],
    )
    def k(t_h, idx_h, o_h, vidx, buf):
        cid = lax.axis_index("c")
        sid = lax.axis_index("s")
        wid = sid * 2 + cid
        pltpu.sync_copy(idx_h.at[wid], vidx)

        @pl.loop(0, KB)
        def _(j):
            base = wid * (KB * BE) + j * BE
            pltpu.sync_copy(t_h.at[vidx.at[j]], buf)
            pltpu.sync_copy(buf, o_h.at[pl.ds(base, BE)])

    return k(tbl, idx3)


def _sc_scatter_add(vals, dst3, init2):
    W = vals.shape[1]

    @functools.partial(
        pl.kernel,
        mesh=_sc_mesh(),
        out_type=jax.ShapeDtypeStruct((2, NPAD, W), jnp.float32),
        compiler_params=_SC_PARAMS,
        scratch_types=[
            pltpu.VMEM((KB, BE), jnp.int32),
            pltpu.VMEM((BE, W), jnp.float32),
            pltpu.VMEM_SHARED((NPAD, W), jnp.float32),
        ],
    )
    def k(v_h, dst_h, init_h, out_h, didx, buf, acc):
        cid = lax.axis_index("c")
        sid = lax.axis_index("s")
        wid = sid * 2 + cid

        @pl.when(sid == 0)
        def _():
            pltpu.sync_copy(init_h.at[cid], acc)

        plsc.subcore_barrier()
        pltpu.sync_copy(dst_h.at[wid], didx)

        @pl.loop(0, KB)
        def _(j):
            base = wid * (KB * BE) + j * BE
            pltpu.sync_copy(v_h.at[pl.ds(base, BE)], buf)
            pltpu.sync_copy(buf, acc.at[didx.at[j]], add=True)

        plsc.subcore_barrier()

        @pl.when(sid == 0)
        def _():
            pltpu.sync_copy(acc, out_h.at[cid])

    return k(vals, dst3, init2)


def _sc_gather_scatter_add(table, src3, dst3, init2):
    W = table.shape[1]

    @functools.partial(
        pl.kernel,
        mesh=_sc_mesh(),
        out_type=jax.ShapeDtypeStruct((2, NPAD, W), jnp.float32),
        compiler_params=_SC_PARAMS,
        scratch_types=[
            pltpu.VMEM((KB, BE), jnp.int32),
            pltpu.VMEM((KB, BE), jnp.int32),
            pltpu.VMEM((BE, W), jnp.float32),
            pltpu.VMEM_SHARED((NPAD, W), jnp.float32),
        ],
    )
    def k(tbl_h, src_h, dst_h, init_h, out_h, sidx, didx, buf, acc):
        cid = lax.axis_index("c")
        sid = lax.axis_index("s")
        wid = sid * 2 + cid

        @pl.when(sid == 0)
        def _():
            pltpu.sync_copy(init_h.at[cid], acc)

        plsc.subcore_barrier()
        pltpu.sync_copy(src_h.at[wid], sidx)
        pltpu.sync_copy(dst_h.at[wid], didx)

        @pl.loop(0, KB)
        def _(j):
            pltpu.sync_copy(tbl_h.at[sidx.at[j]], buf)
            pltpu.sync_copy(buf, acc.at[didx.at[j]], add=True)

        plsc.subcore_barrier()

        @pl.when(sid == 0)
        def _():
            pltpu.sync_copy(acc, out_h.at[cid])

    return k(table, src3, dst3, init2)


def _tc_alpha(xlg, xrg, ep, attf, M):
    def body(xl_ref, xr_ref, ep_ref, at_ref, m_ref, o_ref):
        z = xl_ref[...] + xr_ref[...] + ep_ref[...]
        m = jnp.where(z >= 0, z, 0.2 * z)
        s = m * at_ref[...]
        alpha = jnp.dot(s, m_ref[...], preferred_element_type=jnp.float32)
        o_ref[...] = jnp.exp(alpha)

    B = 2048
    return pl.pallas_call(
        body,
        grid=(EPAD // B,),
        in_specs=[
            pl.BlockSpec((B, WF), lambda i: (i, 0)),
            pl.BlockSpec((B, WF), lambda i: (i, 0)),
            pl.BlockSpec((B, WF), lambda i: (i, 0)),
            pl.BlockSpec((1, WF), lambda i: (0, 0)),
            pl.BlockSpec((WF, 16), lambda i: (0, 0)),
        ],
        out_specs=pl.BlockSpec((B, 16), lambda i: (i, 0)),
        out_shape=jax.ShapeDtypeStruct((EPAD, 16), jnp.float32),
    )(xlg, xrg, ep, attf, M)


def _tc_loop_alpha(xl, xr, ea_mean, WeT, attf, M):
    def body(xl_ref, xr_ref, eam_ref, we_ref, at_ref, m_ref, o_ref):
        epm = jnp.dot(eam_ref[...], we_ref[...],
                      preferred_element_type=jnp.float32)
        z = xl_ref[...] + xr_ref[...] + epm
        m = jnp.where(z >= 0, z, 0.2 * z)
        s = m * at_ref[...]
        o_ref[...] = jnp.exp(
            jnp.dot(s, m_ref[...], preferred_element_type=jnp.float32))

    B = 2504
    return pl.pallas_call(
        body,
        grid=(NPAD // B,),
        in_specs=[
            pl.BlockSpec((B, WF), lambda i: (i, 0)),
            pl.BlockSpec((B, WF), lambda i: (i, 0)),
            pl.BlockSpec((1, 8), lambda i: (0, 0)),
            pl.BlockSpec((8, WF), lambda i: (0, 0)),
            pl.BlockSpec((1, WF), lambda i: (0, 0)),
            pl.BlockSpec((WF, 16), lambda i: (0, 0)),
        ],
        out_specs=pl.BlockSpec((B, 16), lambda i: (i, 0)),
        out_shape=jax.ShapeDtypeStruct((NPAD, 16), jnp.float32),
    )(xl, xr, ea_mean, WeT, attf, M)


def _tc_ea_mean(ea):
    def body(ea_ref, o_ref):
        @pl.when(pl.program_id(0) == 0)
        def _():
            o_ref[...] = jnp.zeros_like(o_ref)
        o_ref[...] += jnp.sum(ea_ref[...], axis=0, keepdims=True) / E

    B = 2000
    return pl.pallas_call(
        body,
        grid=(E // B,),
        in_specs=[pl.BlockSpec((B, 8), lambda i: (i, 0))],
        out_specs=pl.BlockSpec((1, 8), lambda i: (0, 0)),
        out_shape=jax.ShapeDtypeStruct((1, 8), jnp.float32),
    )(ea)


def _tc_den(parts):
    def body(p_ref, o_ref):
        o_ref[...] = p_ref[0] + p_ref[1]

    return pl.pallas_call(
        body,
        in_specs=[pl.BlockSpec((2, NPAD, 16), lambda: (0, 0, 0))],
        out_specs=pl.BlockSpec((NPAD, 16), lambda: (0, 0)),
        out_shape=jax.ShapeDtypeStruct((NPAD, 16), jnp.float32),
    )(parts)


def _tc_q(xlg, exa, deng, MT, R):
    def body(xl_ref, ex_ref, dn_ref, mt_ref, r_ref, o_ref):
        a = ex_ref[...] / (dn_ref[...] + 1e-16)
        arep = jnp.dot(a, mt_ref[...], preferred_element_type=jnp.float32)
        w = arep * xl_ref[...]
        o_ref[...] = jnp.dot(w, r_ref[...], preferred_element_type=jnp.float32)

    B = 2048
    return pl.pallas_call(
        body,
        grid=(EPAD // B,),
        in_specs=[
            pl.BlockSpec((B, WF), lambda i: (i, 0)),
            pl.BlockSpec((B, 16), lambda i: (i, 0)),
            pl.BlockSpec((B, 16), lambda i: (i, 0)),
            pl.BlockSpec((16, WF), lambda i: (0, 0)),
            pl.BlockSpec((WF, 32), lambda i: (0, 0)),
        ],
        out_specs=pl.BlockSpec((B, 32), lambda i: (i, 0)),
        out_shape=jax.ShapeDtypeStruct((EPAD, 32), jnp.float32),
    )(xlg, exa, deng, MT, R)


def _tc_qloop(xl, exl, den, MT, R):
    def body(xl_ref, ex_ref, dn_ref, mt_ref, r_ref, o_ref):
        a = ex_ref[...] / (dn_ref[...] + 1e-16)
        arep = jnp.dot(a, mt_ref[...], preferred_element_type=jnp.float32)
        w = arep * xl_ref[...]
        o_ref[...] = jnp.dot(w, r_ref[...], preferred_element_type=jnp.float32)

    B = 2504
    return pl.pallas_call(
        body,
        grid=(NPAD // B,),
        in_specs=[
            pl.BlockSpec((B, WF), lambda i: (i, 0)),
            pl.BlockSpec((B, 16), lambda i: (i, 0)),
            pl.BlockSpec((B, 16), lambda i: (i, 0)),
            pl.BlockSpec((16, WF), lambda i: (0, 0)),
            pl.BlockSpec((WF, 32), lambda i: (0, 0)),
        ],
        out_specs=pl.BlockSpec((B, 32), lambda i: (i, 0)),
        out_shape=jax.ShapeDtypeStruct((NPAD, 32), jnp.float32),
    )(xl, exl, den, MT, R)


def _tc_m1(qparts, bias32, ggcWT):
    def body(q_ref, b_ref, w_ref, m_ref, mw_ref):
        m1 = (q_ref[0] + q_ref[1]) * (1.0 / HEADS) + b_ref[...]
        m_ref[...] = m1
        mw_ref[...] = jnp.dot(m1, w_ref[...],
                              preferred_element_type=jnp.float32)

    return pl.pallas_call(
        body,
        in_specs=[
            pl.BlockSpec((2, NPAD, 32), lambda: (0, 0, 0)),
            pl.BlockSpec((1, 32), lambda: (0, 0)),
            pl.BlockSpec((32, 32), lambda: (0, 0)),
        ],
        out_specs=[
            pl.BlockSpec((NPAD, 32), lambda: (0, 0)),
            pl.BlockSpec((NPAD, 32), lambda: (0, 0)),
        ],
        out_shape=[
            jax.ShapeDtypeStruct((NPAD, 32), jnp.float32),
            jax.ShapeDtypeStruct((NPAD, 32), jnp.float32),
        ],
    )(qparts, bias32, ggcWT)


def _tc_gru(mparts, m1, WihT, bih, WhhT, bhh):
    def body(p_ref, x_ref, wi_ref, bi_ref, wh_ref, bh_ref, o_ref):
        agg = p_ref[0] + p_ref[1]
        x = x_ref[...]
        gi = jnp.dot(agg, wi_ref[...],
                     preferred_element_type=jnp.float32) + bi_ref[...]
        gh = jnp.dot(x, wh_ref[...],
                     preferred_element_type=jnp.float32) + bh_ref[...]
        r = jax.nn.sigmoid(gi[:, 0:D] + gh[:, 0:D])
        z = jax.nn.sigmoid(gi[:, D:2 * D] + gh[:, D:2 * D])
        nt = jnp.tanh(gi[:, 2 * D:3 * D] + r * gh[:, 2 * D:3 * D])
        hn = (1.0 - z) * nt + z * x[:, 0:D]
        o_ref[...] = jnp.pad(hn, ((0, 0), (0, 12)))

    return pl.pallas_call(
        body,
        in_specs=[
            pl.BlockSpec((2, NPAD, 32), lambda: (0, 0, 0)),
            pl.BlockSpec((NPAD, 32), lambda: (0, 0)),
            pl.BlockSpec((32, 64), lambda: (0, 0)),
            pl.BlockSpec((1, 64), lambda: (0, 0)),
            pl.BlockSpec((32, 64), lambda: (0, 0)),
            pl.BlockSpec((1, 64), lambda: (0, 0)),
        ],
        out_specs=pl.BlockSpec((NPAD, 32), lambda: (0, 0)),
        out_shape=jax.ShapeDtypeStruct((NPAD, 32), jnp.float32),
    )(mparts, m1, WihT, bih, WhhT, bhh)


def _tc_score(hparts, H, WrelT, brel, WrootT):
    def body(p_ref, h_ref, wr_ref, br_ref, wo_ref, o_ref):
        agg = p_ref[0] + p_ref[1]
        o_ref[...] = (
            jnp.dot(agg, wr_ref[...], preferred_element_type=jnp.float32)
            + br_ref[...]
            + jnp.dot(h_ref[...], wo_ref[...],
                      preferred_element_type=jnp.float32)
        )

    return pl.pallas_call(
        body,
        in_specs=[
            pl.BlockSpec((2, NPAD, 64), lambda: (0, 0, 0)),
            pl.BlockSpec((NPAD, 64), lambda: (0, 0)),
            pl.BlockSpec((64, 8), lambda: (0, 0)),
            pl.BlockSpec((1, 8), lambda: (0, 0)),
            pl.BlockSpec((64, 8), lambda: (0, 0)),
        ],
        out_specs=pl.BlockSpec((NPAD, 8), lambda: (0, 0)),
        out_shape=jax.ShapeDtypeStruct((NPAD, 8), jnp.float32),
    )(hparts, H, WrelT, brel, WrootT)


def _tc_rank(scoreC, scoreR, batchC, batchR):
    BI, BJ = 512, 2048

    def body(si_ref, sj_ref, bi_ref, bj_ref, r_ref, c_ref):
        i0 = pl.program_id(0) * BI
        j0 = pl.program_id(1) * BJ

        @pl.when(pl.program_id(1) == 0)
        def _():
            r_ref[...] = jnp.zeros_like(r_ref)
            c_ref[...] = jnp.zeros_like(c_ref)

        ii = i0 + lax.broadcasted_iota(jnp.int32, (BI, BJ), 0)
        jj = j0 + lax.broadcasted_iota(jnp.int32, (BI, BJ), 1)
        eq = bi_ref[...] == bj_ref[...]
        sj = sj_ref[...]
        si = si_ref[...]
        ahead = (sj > si) | ((sj == si) & (jj < ii))
        contrib = jnp.where(eq & ahead, 1.0, 0.0)
        cgrp = jnp.where(eq, 1.0, 0.0)
        r_ref[...] += jnp.sum(contrib, axis=1, keepdims=True)
        c_ref[...] += jnp.sum(cgrp, axis=1, keepdims=True)

    return pl.pallas_call(
        body,
        grid=(NP2 // BI, NP2 // BJ),
        in_specs=[
            pl.BlockSpec((BI, 1), lambda i, j: (i, 0)),
            pl.BlockSpec((1, BJ), lambda i, j: (0, j)),
            pl.BlockSpec((BI, 1), lambda i, j: (i, 0)),
            pl.BlockSpec((1, BJ), lambda i, j: (0, j)),
        ],
        out_specs=[
            pl.BlockSpec((BI, 1), lambda i, j: (i, 0)),
            pl.BlockSpec((BI, 1), lambda i, j: (i, 0)),
        ],
        out_shape=[
            jax.ShapeDtypeStruct((NP2, 1), jnp.float32),
            jax.ShapeDtypeStruct((NP2, 1), jnp.float32),
        ],
    )(scoreC, scoreR, batchC, batchR)


def _tc_pool(H, score, rank, cnt, batchC):
    B = 1024

    def body(h_ref, s_ref, r_ref, c_ref, b_ref, o_ref):
        @pl.when(pl.program_id(0) == 0)
        def _():
            o_ref[...] = jnp.zeros_like(o_ref)

        kq = jnp.ceil(0.3 * c_ref[...])
        mask = jnp.where(r_ref[...] < kq, 1.0, 0.0)
        hs = h_ref[...] * jnp.tanh(s_ref[...]) * mask
        gid = lax.broadcasted_iota(jnp.int32, (B, 64), 1)
        oh = jnp.where(b_ref[...] == gid, 1.0, 0.0)
        o_ref[...] += lax.dot_general(
            oh, hs, (((0,), (0,)), ((), ())),
            preferred_element_type=jnp.float32)

    return pl.pallas_call(
        body,
        grid=(NP2 // B,),
        in_specs=[
            pl.BlockSpec((B, 64), lambda i: (i, 0)),
            pl.BlockSpec((B, 1), lambda i: (i, 0)),
            pl.BlockSpec((B, 1), lambda i: (i, 0)),
            pl.BlockSpec((B, 1), lambda i: (i, 0)),
            pl.BlockSpec((B, 1), lambda i: (i, 0)),
        ],
        out_specs=pl.BlockSpec((64, 64), lambda i: (0, 0)),
        out_shape=jax.ShapeDtypeStruct((64, 64), jnp.float32),
    )(H, score, rank, cnt, batchC)


def _tc_mlp(g, W1T, b1, W2T, b2, WoT, bo):
    def body(g_ref, w1, b1r, w2, b2r, wo, bor, o_ref):
        a = jnp.dot(g_ref[...], w1[...],
                    preferred_element_type=jnp.float32) + b1r[...]
        a = jnp.where(a >= 0, a, 0.01 * a)
        a = jnp.dot(a, w2[...], preferred_element_type=jnp.float32) + b2r[...]
        a = jnp.where(a >= 0, a, 0.01 * a)
        o_ref[...] = jnp.dot(a, wo[...],
                             preferred_element_type=jnp.float32) + bor[...]

    return pl.pallas_call(
        body,
        in_specs=[
            pl.BlockSpec((64, 64), lambda: (0, 0)),
            pl.BlockSpec((64, 64), lambda: (0, 0)),
            pl.BlockSpec((1, 64), lambda: (0, 0)),
            pl.BlockSpec((64, 32), lambda: (0, 0)),
            pl.BlockSpec((1, 32), lambda: (0, 0)),
            pl.BlockSpec((32, 8), lambda: (0, 0)),
            pl.BlockSpec((1, 8), lambda: (0, 0)),
        ],
        out_specs=pl.BlockSpec((64, 8), lambda: (0, 0)),
        out_shape=jax.ShapeDtypeStruct((64, 8), jnp.float32),
    )(g, W1T, b1, W2T, b2, WoT, bo)


def _padw(a, rows, cols):
    return jnp.pad(a, ((0, rows - a.shape[0]), (0, cols - a.shape[1])))


def kernel(x, edge_index, edge_attr, batch, params):
    p = params
    f = jnp.arange(WF)
    valid = (f < HEADS * D)
    M = ((f[:, None] // D == jnp.arange(16)[None, :]) &
         valid[:, None]).astype(jnp.float32)
    MT = M.T
    R = ((f[:, None] % D == jnp.arange(32)[None, :]) &
         valid[:, None]).astype(jnp.float32)
    attf = jnp.pad(p['gat_att'].reshape(1, HEADS * D), ((0, 0), (0, 8)))

    WlT = _padw(p['gat_Wl'].T, 32, WF)
    WrT = _padw(p['gat_Wr'].T, 32, WF)
    WeT = jnp.pad(p['gat_We'].T, ((0, 0), (0, 8)))
    bl = jnp.pad(p['gat_bl'], (0, 8))
    br = jnp.pad(p['gat_br'], (0, 8))
    bias32 = jnp.pad(p['gat_bias'], (0, 12)).reshape(1, 32)
    ggcWT = _padw(p['ggc_weight'], 32, 32)
    WihT = _padw(p['gru_Wih'].T, 32, 64)
    bih = jnp.pad(p['gru_bih'], (0, 4)).reshape(1, 64)
    WhhT = _padw(p['gru_Whh'].T, 32, 64)
    bhh = jnp.pad(p['gru_bhh'], (0, 4)).reshape(1, 64)
    WrelT = _padw(p['pool_Wrel'].T, 64, 8)
    brel = jnp.pad(p['pool_brel'], (0, 7)).reshape(1, 8)
    WrootT = _padw(p['pool_Wroot'].T, 64, 8)
    W1T = _padw(p['fc1_W'].T, 64, 64)
    b1 = jnp.pad(p['fc1_b'], (0, 24)).reshape(1, 64)
    W2T = _padw(p['fc2_W'].T, 64, 32)
    b2 = jnp.pad(p['fc2_b'], (0, 2)).reshape(1, 32)
    WoT = _padw(p['out_W'].T, 32, 8)
    bo = jnp.pad(p['out_b'], (0, 7)).reshape(1, 8)

    src3 = jnp.concatenate(
        [edge_index[0], jnp.zeros((EPAD - E,), jnp.int32)]).reshape(NW, KB, BE)
    dst3 = jnp.concatenate(
        [edge_index[1], jnp.full((EPAD - E,), N, jnp.int32)]).reshape(NW, KB, BE)

    x32 = _padw(x, NPAD, 32)
    ea_pad = jnp.pad(edge_attr, ((0, EPAD - E), (0, 0)))
    ep = _linear(ea_pad, WeT, jnp.zeros((WF,), jnp.float32), 2048)
    ea_mean = _tc_ea_mean(edge_attr)

    zeros32 = jnp.zeros((2, NPAD, 32), jnp.float32)
    z16 = jnp.zeros((NPAD, 16), jnp.float32)
    z32 = jnp.zeros((NPAD, 32), jnp.float32)

    def gat_layer(h32):
        xl = _linear(h32, WlT, bl, 2504)
        xr = _linear(h32, WrT, br, 2504)
        xlg, xrg = _sc_gather2(xl, xr, src3, dst3)
        exa = _tc_alpha(xlg, xrg, ep, attf, M)
        exl = _tc_loop_alpha(xl, xr, ea_mean, WeT, attf, M)
        denp = _sc_scatter_add(exa, dst3, jnp.stack([exl, z16]))
        den = _tc_den(denp)
        deng = _sc_gather1(den, dst3)
        q = _tc_q(xlg, exa, deng, MT, R)
        qloop = _tc_qloop(xl, exl, den, MT, R)
        qparts = _sc_scatter_add(q, dst3, jnp.stack([qloop, z32]))
        m1, mW = _tc_m1(qparts, bias32, ggcWT)
        aggm = _sc_gather_scatter_add(mW, src3, dst3, zeros32)
        return _tc_gru(aggm, m1, WihT, bih, WhhT, bhh)

    h1 = gat_layer(x32)
    h2 = gat_layer(h1)

    H = jnp.concatenate([x32[:, :D], h1[:, :D], h2[:, :D]], axis=1)
    H = jnp.pad(H, ((0, 0), (0, 4)))
    hparts = _sc_gather_scatter_add(
        H, src3, dst3, jnp.zeros((2, NPAD, 64), jnp.float32))
    score = _tc_score(hparts, H, WrelT, brel, WrootT)[:, 0:1]

    scoreC = jnp.concatenate(
        [score[:N], jnp.zeros((NP2 - N, 1), jnp.float32)])
    batchC = jnp.concatenate(
        [batch, jnp.full((NP2 - N,), N_GRAPHS, jnp.int32)]).reshape(NP2, 1)
    scoreR = scoreC.reshape(1, NP2)
    batchR = batchC.reshape(1, NP2)
    rank, cnt = _tc_rank(scoreC, scoreR, batchC, batchR)

    H2 = jnp.pad(H[:N], ((0, NP2 - N), (0, 0)))
    g = _tc_pool(H2, scoreC, rank, cnt, batchC)
    out = _tc_mlp(g, W1T, b1, W2T, b2, WoT, bo)
    return out[:, 0]


# speedup vs baseline: 14.6929x; 1.1050x over previous
import functools

import jax
import jax.numpy as jnp
from jax import lax
from jax.experimental import pallas as pl
from jax.experimental.pallas import tpu as pltpu
from jax.experimental.pallas import tpu_sc as plsc

HEADS = 10
D = 20
N_GRAPHS = 64
N = 10000
E = 160000
NPAD = 10016
NW = 32
BE = 80
EPAD = 163840
KB = EPAD // (NW * BE)
KB2 = 2 * KB
NB = 4
WF = 208
NP2 = 10240

_SC_PARAMS = pltpu.CompilerParams(use_tc_tiling_on_sc=False)


def _linear_block(x_ref, w_ref, b_ref, o_ref):
    o_ref[...] = (
        jnp.dot(x_ref[...], w_ref[...], preferred_element_type=jnp.float32)
        + b_ref[...]
    )


def _linear(x, Wt, b, block_rows):
    n, k = x.shape
    m = Wt.shape[1]
    b2 = b.reshape(1, m)
    return pl.pallas_call(
        _linear_block,
        grid=(n // block_rows,),
        in_specs=[
            pl.BlockSpec((block_rows, k), lambda i: (i, 0)),
            pl.BlockSpec((k, m), lambda i: (0, 0)),
            pl.BlockSpec((1, m), lambda i: (0, 0)),
        ],
        out_specs=pl.BlockSpec((block_rows, m), lambda i: (i, 0)),
        out_shape=jax.ShapeDtypeStruct((n, m), jnp.float32),
    )(x, Wt, b2)


def _sc_mesh():
    return plsc.VectorSubcoreMesh(core_axis_name="c", subcore_axis_name="s")


def _sc_gather_all(tbl, idx3):
    W = tbl.shape[1]
    NE = NW * KB2 * BE

    @functools.partial(
        pl.kernel,
        mesh=_sc_mesh(),
        out_type=jax.ShapeDtypeStruct((NE, W), jnp.float32),
        compiler_params=_SC_PARAMS,
        scratch_types=[
            pltpu.VMEM((KB2, BE), jnp.int32),
            pltpu.VMEM((NB, BE, W), jnp.float32),
        ] + [pltpu.SemaphoreType.DMA] * NB,
    )
    def k(t_h, idx_h, o_h, vidx, bufs, *sems):
        cid = lax.axis_index("c")
        sid = lax.axis_index("s")
        wid = sid * 2 + cid
        pltpu.sync_copy(idx_h.at[wid], vidx)

        @pl.loop(0, KB2, step=NB)
        def _(g):
            hs = [pltpu.async_copy(t_h.at[vidx.at[g + b]], bufs.at[b], sems[b])
                  for b in range(NB)]
            for b in range(NB):
                hs[b].wait()
                pltpu.sync_copy(
                    bufs.at[b], o_h.at[pl.ds(wid * (KB2 * BE) + (g + b) * BE, BE)])

    return k(tbl, idx3)


def _sc_gather1(tbl, idx3):
    W = tbl.shape[1]

    @functools.partial(
        pl.kernel,
        mesh=_sc_mesh(),
        out_type=jax.ShapeDtypeStruct((EPAD, W), jnp.float32),
        compiler_params=_SC_PARAMS,
        scratch_types=[
            pltpu.VMEM((KB, BE), jnp.int32),
            pltpu.VMEM((NB, BE, W), jnp.float32),
        ] + [pltpu.SemaphoreType.DMA] * NB,
    )
    def k(t_h, idx_h, o_h, vidx, bufs, *sems):
        cid = lax.axis_index("c")
        sid = lax.axis_index("s")
        wid = sid * 2 + cid
        pltpu.sync_copy(idx_h.at[wid], vidx)

        @pl.loop(0, KB, step=NB)
        def _(g):
            hs = [pltpu.async_copy(t_h.at[vidx.at[g + b]], bufs.at[b], sems[b])
                  for b in range(NB)]
            for b in range(NB):
                hs[b].wait()
                pltpu.sync_copy(
                    bufs.at[b], o_h.at[pl.ds(wid * (KB * BE) + (g + b) * BE, BE)])

    return k(tbl, idx3)


def _sc_scatter_add(vals, dst3, init2):
    W = vals.shape[1]

    @functools.partial(
        pl.kernel,
        mesh=_sc_mesh(),
        out_type=jax.ShapeDtypeStruct((2, NPAD, W), jnp.float32),
        compiler_params=_SC_PARAMS,
        scratch_types=[
            pltpu.VMEM((KB, BE), jnp.int32),
            pltpu.VMEM((NB, BE, W), jnp.float32),
            pltpu.VMEM_SHARED((NPAD, W), jnp.float32),
        ] + [pltpu.SemaphoreType.DMA] * NB,
    )
    def k(v_h, dst_h, init_h, out_h, didx, bufs, acc, *sems):
        cid = lax.axis_index("c")
        sid = lax.axis_index("s")
        wid = sid * 2 + cid

        @pl.when(sid == 0)
        def _():
            pltpu.sync_copy(init_h.at[cid], acc)

        plsc.subcore_barrier()
        pltpu.sync_copy(dst_h.at[wid], didx)

        @pl.loop(0, KB, step=NB)
        def _(g):
            hs = [pltpu.async_copy(
                v_h.at[pl.ds(wid * (KB * BE) + (g + b) * BE, BE)],
                bufs.at[b], sems[b]) for b in range(NB)]
            for b in range(NB):
                hs[b].wait()
                pltpu.sync_copy(bufs.at[b], acc.at[didx.at[g + b]], add=True)

        plsc.subcore_barrier()

        @pl.when(sid == 0)
        def _():
            pltpu.sync_copy(acc, out_h.at[cid])

    return k(vals, dst3, init2)


def _sc_gather_scatter_add(table, src3, dst3, init2):
    W = table.shape[1]

    @functools.partial(
        pl.kernel,
        mesh=_sc_mesh(),
        out_type=jax.ShapeDtypeStruct((2, NPAD, W), jnp.float32),
        compiler_params=_SC_PARAMS,
        scratch_types=[
            pltpu.VMEM((KB, BE), jnp.int32),
            pltpu.VMEM((KB, BE), jnp.int32),
            pltpu.VMEM((NB, BE, W), jnp.float32),
            pltpu.VMEM_SHARED((NPAD, W), jnp.float32),
        ] + [pltpu.SemaphoreType.DMA] * NB,
    )
    def k(tbl_h, src_h, dst_h, init_h, out_h, sidx, didx, bufs, acc, *sems):
        cid = lax.axis_index("c")
        sid = lax.axis_index("s")
        wid = sid * 2 + cid

        @pl.when(sid == 0)
        def _():
            pltpu.sync_copy(init_h.at[cid], acc)

        plsc.subcore_barrier()
        pltpu.sync_copy(src_h.at[wid], sidx)
        pltpu.sync_copy(dst_h.at[wid], didx)

        @pl.loop(0, KB, step=NB)
        def _(g):
            hs = [pltpu.async_copy(t_h_at(tbl_h, sidx, g + b), bufs.at[b],
                                   sems[b]) for b in range(NB)]
            for b in range(NB):
                hs[b].wait()
                pltpu.sync_copy(bufs.at[b], acc.at[didx.at[g + b]], add=True)

        plsc.subcore_barrier()

        @pl.when(sid == 0)
        def _():
            pltpu.sync_copy(acc, out_h.at[cid])

    return k(table, src3, dst3, init2)


def t_h_at(tbl_h, sidx, j):
    return tbl_h.at[sidx.at[j]]


def _tc_alpha(gAll, ep, attf, M):
    def body(xl_ref, xr_ref, ep_ref, at_ref, m_ref, o_ref):
        z = xl_ref[...] + xr_ref[...] + ep_ref[...]
        m = jnp.where(z >= 0, z, 0.2 * z)
        s = m * at_ref[...]
        alpha = jnp.dot(s, m_ref[...], preferred_element_type=jnp.float32)
        o_ref[...] = jnp.exp(alpha)

    B = 2048
    nblk = EPAD // B
    return pl.pallas_call(
        body,
        grid=(nblk,),
        in_specs=[
            pl.BlockSpec((B, WF), lambda i: (i, 0)),
            pl.BlockSpec((B, WF), lambda i: (i + nblk, 0)),
            pl.BlockSpec((B, WF), lambda i: (i, 0)),
            pl.BlockSpec((1, WF), lambda i: (0, 0)),
            pl.BlockSpec((WF, 16), lambda i: (0, 0)),
        ],
        out_specs=pl.BlockSpec((B, 16), lambda i: (i, 0)),
        out_shape=jax.ShapeDtypeStruct((EPAD, 16), jnp.float32),
    )(gAll, gAll, ep, attf, M)


def _tc_loop_alpha(T2v, ea_mean, WeT, attf, M):
    def body(t_ref, eam_ref, we_ref, at_ref, m_ref, o_ref):
        epm = jnp.dot(eam_ref[...], we_ref[...],
                      preferred_element_type=jnp.float32)
        z = t_ref[:, :WF] + t_ref[:, WF:] + epm
        m = jnp.where(z >= 0, z, 0.2 * z)
        s = m * at_ref[...]
        o_ref[...] = jnp.exp(
            jnp.dot(s, m_ref[...], preferred_element_type=jnp.float32))

    B = 2504
    return pl.pallas_call(
        body,
        grid=(NPAD // B,),
        in_specs=[
            pl.BlockSpec((B, 2 * WF), lambda i: (i, 0)),
            pl.BlockSpec((1, 8), lambda i: (0, 0)),
            pl.BlockSpec((8, WF), lambda i: (0, 0)),
            pl.BlockSpec((1, WF), lambda i: (0, 0)),
            pl.BlockSpec((WF, 16), lambda i: (0, 0)),
        ],
        out_specs=pl.BlockSpec((B, 16), lambda i: (i, 0)),
        out_shape=jax.ShapeDtypeStruct((NPAD, 16), jnp.float32),
    )(T2v, ea_mean, WeT, attf, M)


def _tc_ea_mean(ea):
    def body(ea_ref, o_ref):
        @pl.when(pl.program_id(0) == 0)
        def _():
            o_ref[...] = jnp.zeros_like(o_ref)
        o_ref[...] += jnp.sum(ea_ref[...], axis=0, keepdims=True) / E

    B = 2000
    return pl.pallas_call(
        body,
        grid=(E // B,),
        in_specs=[pl.BlockSpec((B, 8), lambda i: (i, 0))],
        out_specs=pl.BlockSpec((1, 8), lambda i: (0, 0)),
        out_shape=jax.ShapeDtypeStruct((1, 8), jnp.float32),
    )(ea)


def _tc_den(parts):
    def body(p_ref, o_ref):
        o_ref[...] = p_ref[0] + p_ref[1]

    return pl.pallas_call(
        body,
        in_specs=[pl.BlockSpec((2, NPAD, 16), lambda: (0, 0, 0))],
        out_specs=pl.BlockSpec((NPAD, 16), lambda: (0, 0)),
        out_shape=jax.ShapeDtypeStruct((NPAD, 16), jnp.float32),
    )(parts)


def _tc_q(xlg, exa, deng, MT, R):
    def body(xl_ref, ex_ref, dn_ref, mt_ref, r_ref, o_ref):
        a = ex_ref[...] / (dn_ref[...] + 1e-16)
        arep = jnp.dot(a, mt_ref[...], preferred_element_type=jnp.float32)
        w = arep * xl_ref[...]
        o_ref[...] = jnp.dot(w, r_ref[...], preferred_element_type=jnp.float32)

    B = 2048
    return pl.pallas_call(
        body,
        grid=(EPAD // B,),
        in_specs=[
            pl.BlockSpec((B, WF), lambda i: (i, 0)),
            pl.BlockSpec((B, 16), lambda i: (i, 0)),
            pl.BlockSpec((B, 16), lambda i: (i, 0)),
            pl.BlockSpec((16, WF), lambda i: (0, 0)),
            pl.BlockSpec((WF, 32), lambda i: (0, 0)),
        ],
        out_specs=pl.BlockSpec((B, 32), lambda i: (i, 0)),
        out_shape=jax.ShapeDtypeStruct((EPAD, 32), jnp.float32),
    )(xlg, exa, deng, MT, R)


def _tc_qloop(xl, exl, den, MT, R):
    def body(xl_ref, ex_ref, dn_ref, mt_ref, r_ref, o_ref):
        a = ex_ref[...] / (dn_ref[...] + 1e-16)
        arep = jnp.dot(a, mt_ref[...], preferred_element_type=jnp.float32)
        w = arep * xl_ref[:, :WF]
        o_ref[...] = jnp.dot(w, r_ref[...], preferred_element_type=jnp.float32)

    B = 2504
    return pl.pallas_call(
        body,
        grid=(NPAD // B,),
        in_specs=[
            pl.BlockSpec((B, 2 * WF), lambda i: (i, 0)),
            pl.BlockSpec((B, 16), lambda i: (i, 0)),
            pl.BlockSpec((B, 16), lambda i: (i, 0)),
            pl.BlockSpec((16, WF), lambda i: (0, 0)),
            pl.BlockSpec((WF, 32), lambda i: (0, 0)),
        ],
        out_specs=pl.BlockSpec((B, 32), lambda i: (i, 0)),
        out_shape=jax.ShapeDtypeStruct((NPAD, 32), jnp.float32),
    )(xl, exl, den, MT, R)


def _tc_m1(qparts, bias32, ggcWT):
    def body(q_ref, b_ref, w_ref, m_ref, mw_ref):
        m1 = (q_ref[0] + q_ref[1]) * (1.0 / HEADS) + b_ref[...]
        m_ref[...] = m1
        mw_ref[...] = jnp.dot(m1, w_ref[...],
                              preferred_element_type=jnp.float32)

    return pl.pallas_call(
        body,
        in_specs=[
            pl.BlockSpec((2, NPAD, 32), lambda: (0, 0, 0)),
            pl.BlockSpec((1, 32), lambda: (0, 0)),
            pl.BlockSpec((32, 32), lambda: (0, 0)),
        ],
        out_specs=[
            pl.BlockSpec((NPAD, 32), lambda: (0, 0)),
            pl.BlockSpec((NPAD, 32), lambda: (0, 0)),
        ],
        out_shape=[
            jax.ShapeDtypeStruct((NPAD, 32), jnp.float32),
            jax.ShapeDtypeStruct((NPAD, 32), jnp.float32),
        ],
    )(qparts, bias32, ggcWT)


def _tc_gru(mparts, m1, WihT, bih, WhhT, bhh):
    def body(p_ref, x_ref, wi_ref, bi_ref, wh_ref, bh_ref, o_ref):
        agg = p_ref[0] + p_ref[1]
        x = x_ref[...]
        gi = jnp.dot(agg, wi_ref[...],
                     preferred_element_type=jnp.float32) + bi_ref[...]
        gh = jnp.dot(x, wh_ref[...],
                     preferred_element_type=jnp.float32) + bh_ref[...]
        r = jax.nn.sigmoid(gi[:, 0:D] + gh[:, 0:D])
        z = jax.nn.sigmoid(gi[:, D:2 * D] + gh[:, D:2 * D])
        nt = jnp.tanh(gi[:, 2 * D:3 * D] + r * gh[:, 2 * D:3 * D])
        hn = (1.0 - z) * nt + z * x[:, 0:D]
        o_ref[...] = jnp.pad(hn, ((0, 0), (0, 12)))

    return pl.pallas_call(
        body,
        in_specs=[
            pl.BlockSpec((2, NPAD, 32), lambda: (0, 0, 0)),
            pl.BlockSpec((NPAD, 32), lambda: (0, 0)),
            pl.BlockSpec((32, 64), lambda: (0, 0)),
            pl.BlockSpec((1, 64), lambda: (0, 0)),
            pl.BlockSpec((32, 64), lambda: (0, 0)),
            pl.BlockSpec((1, 64), lambda: (0, 0)),
        ],
        out_specs=pl.BlockSpec((NPAD, 32), lambda: (0, 0)),
        out_shape=jax.ShapeDtypeStruct((NPAD, 32), jnp.float32),
    )(mparts, m1, WihT, bih, WhhT, bhh)


def _tc_score(hparts, H, WrelT, brel, WrootT):
    def body(p_ref, h_ref, wr_ref, br_ref, wo_ref, o_ref):
        agg = p_ref[0] + p_ref[1]
        o_ref[...] = (
            jnp.dot(agg, wr_ref[...], preferred_element_type=jnp.float32)
            + br_ref[...]
            + jnp.dot(h_ref[...], wo_ref[...],
                      preferred_element_type=jnp.float32)
        )

    return pl.pallas_call(
        body,
        in_specs=[
            pl.BlockSpec((2, NPAD, 64), lambda: (0, 0, 0)),
            pl.BlockSpec((NPAD, 64), lambda: (0, 0)),
            pl.BlockSpec((64, 8), lambda: (0, 0)),
            pl.BlockSpec((1, 8), lambda: (0, 0)),
            pl.BlockSpec((64, 8), lambda: (0, 0)),
        ],
        out_specs=pl.BlockSpec((NPAD, 8), lambda: (0, 0)),
        out_shape=jax.ShapeDtypeStruct((NPAD, 8), jnp.float32),
    )(hparts, H, WrelT, brel, WrootT)


def _tc_rank(scoreC, scoreR, batchC, batchR):
    BI, BJ = 512, 2048

    def body(si_ref, sj_ref, bi_ref, bj_ref, r_ref, c_ref):
        i0 = pl.program_id(0) * BI
        j0 = pl.program_id(1) * BJ

        @pl.when(pl.program_id(1) == 0)
        def _():
            r_ref[...] = jnp.zeros_like(r_ref)
            c_ref[...] = jnp.zeros_like(c_ref)

        ii = i0 + lax.broadcasted_iota(jnp.int32, (BI, BJ), 0)
        jj = j0 + lax.broadcasted_iota(jnp.int32, (BI, BJ), 1)
        eq = bi_ref[...] == bj_ref[...]
        sj = sj_ref[...]
        si = si_ref[...]
        ahead = (sj > si) | ((sj == si) & (jj < ii))
        contrib = jnp.where(eq & ahead, 1.0, 0.0)
        cgrp = jnp.where(eq, 1.0, 0.0)
        r_ref[...] += jnp.sum(contrib, axis=1, keepdims=True)
        c_ref[...] += jnp.sum(cgrp, axis=1, keepdims=True)

    return pl.pallas_call(
        body,
        grid=(NP2 // BI, NP2 // BJ),
        in_specs=[
            pl.BlockSpec((BI, 1), lambda i, j: (i, 0)),
            pl.BlockSpec((1, BJ), lambda i, j: (0, j)),
            pl.BlockSpec((BI, 1), lambda i, j: (i, 0)),
            pl.BlockSpec((1, BJ), lambda i, j: (0, j)),
        ],
        out_specs=[
            pl.BlockSpec((BI, 1), lambda i, j: (i, 0)),
            pl.BlockSpec((BI, 1), lambda i, j: (i, 0)),
        ],
        out_shape=[
            jax.ShapeDtypeStruct((NP2, 1), jnp.float32),
            jax.ShapeDtypeStruct((NP2, 1), jnp.float32),
        ],
    )(scoreC, scoreR, batchC, batchR)


def _tc_pool(H, score, rank, cnt, batchC):
    B = 1024

    def body(h_ref, s_ref, r_ref, c_ref, b_ref, o_ref):
        @pl.when(pl.program_id(0) == 0)
        def _():
            o_ref[...] = jnp.zeros_like(o_ref)

        kq = jnp.ceil(0.3 * c_ref[...])
        mask = jnp.where(r_ref[...] < kq, 1.0, 0.0)
        hs = h_ref[...] * jnp.tanh(s_ref[...]) * mask
        gid = lax.broadcasted_iota(jnp.int32, (B, 64), 1)
        oh = jnp.where(b_ref[...] == gid, 1.0, 0.0)
        o_ref[...] += lax.dot_general(
            oh, hs, (((0,), (0,)), ((), ())),
            preferred_element_type=jnp.float32)

    return pl.pallas_call(
        body,
        grid=(NP2 // B,),
        in_specs=[
            pl.BlockSpec((B, 64), lambda i: (i, 0)),
            pl.BlockSpec((B, 1), lambda i: (i, 0)),
            pl.BlockSpec((B, 1), lambda i: (i, 0)),
            pl.BlockSpec((B, 1), lambda i: (i, 0)),
            pl.BlockSpec((B, 1), lambda i: (i, 0)),
        ],
        out_specs=pl.BlockSpec((64, 64), lambda i: (0, 0)),
        out_shape=jax.ShapeDtypeStruct((64, 64), jnp.float32),
    )(H, score, rank, cnt, batchC)


def _tc_mlp(g, W1T, b1, W2T, b2, WoT, bo):
    def body(g_ref, w1, b1r, w2, b2r, wo, bor, o_ref):
        a = jnp.dot(g_ref[...], w1[...],
                    preferred_element_type=jnp.float32) + b1r[...]
        a = jnp.where(a >= 0, a, 0.01 * a)
        a = jnp.dot(a, w2[...], preferred_element_type=jnp.float32) + b2r[...]
        a = jnp.where(a >= 0, a, 0.01 * a)
        o_ref[...] = jnp.dot(a, wo[...],
                             preferred_element_type=jnp.float32) + bor[...]

    return pl.pallas_call(
        body,
        in_specs=[
            pl.BlockSpec((64, 64), lambda: (0, 0)),
            pl.BlockSpec((64, 64), lambda: (0, 0)),
            pl.BlockSpec((1, 64), lambda: (0, 0)),
            pl.BlockSpec((64, 32), lambda: (0, 0)),
            pl.BlockSpec((1, 32), lambda: (0, 0)),
            pl.BlockSpec((32, 8), lambda: (0, 0)),
            pl.BlockSpec((1, 8), lambda: (0, 0)),
        ],
        out_specs=pl.BlockSpec((64, 8), lambda: (0, 0)),
        out_shape=jax.ShapeDtypeStruct((64, 8), jnp.float32),
    )(g, W1T, b1, W2T, b2, WoT, bo)


def _padw(a, rows, cols):
    return jnp.pad(a, ((0, rows - a.shape[0]), (0, cols - a.shape[1])))


def kernel(x, edge_index, edge_attr, batch, params):
    p = params
    f = jnp.arange(WF)
    valid = (f < HEADS * D)
    M = ((f[:, None] // D == jnp.arange(16)[None, :]) &
         valid[:, None]).astype(jnp.float32)
    MT = M.T
    R = ((f[:, None] % D == jnp.arange(32)[None, :]) &
         valid[:, None]).astype(jnp.float32)
    attf = jnp.pad(p['gat_att'].reshape(1, HEADS * D), ((0, 0), (0, 8)))

    Wboth = jnp.concatenate(
        [_padw(p['gat_Wl'].T, 32, WF), _padw(p['gat_Wr'].T, 32, WF)], axis=1)
    bboth = jnp.concatenate(
        [jnp.pad(p['gat_bl'], (0, 8)), jnp.pad(p['gat_br'], (0, 8))])
    WeT = jnp.pad(p['gat_We'].T, ((0, 0), (0, 8)))
    bias32 = jnp.pad(p['gat_bias'], (0, 12)).reshape(1, 32)
    ggcWT = _padw(p['ggc_weight'], 32, 32)
    WihT = _padw(p['gru_Wih'].T, 32, 64)
    bih = jnp.pad(p['gru_bih'], (0, 4)).reshape(1, 64)
    WhhT = _padw(p['gru_Whh'].T, 32, 64)
    bhh = jnp.pad(p['gru_bhh'], (0, 4)).reshape(1, 64)
    WrelT = _padw(p['pool_Wrel'].T, 64, 8)
    brel = jnp.pad(p['pool_brel'], (0, 7)).reshape(1, 8)
    WrootT = _padw(p['pool_Wroot'].T, 64, 8)
    W1T = _padw(p['fc1_W'].T, 64, 64)
    b1 = jnp.pad(p['fc1_b'], (0, 24)).reshape(1, 64)
    W2T = _padw(p['fc2_W'].T, 64, 32)
    b2 = jnp.pad(p['fc2_b'], (0, 2)).reshape(1, 32)
    WoT = _padw(p['out_W'].T, 32, 8)
    bo = jnp.pad(p['out_b'], (0, 7)).reshape(1, 8)

    srcP = jnp.concatenate(
        [edge_index[0], jnp.zeros((EPAD - E,), jnp.int32)])
    dstP = jnp.concatenate(
        [edge_index[1], jnp.full((EPAD - E,), N, jnp.int32)])
    src3 = srcP.reshape(NW, KB, BE)
    dst3 = dstP.reshape(NW, KB, BE)
    idxall3 = jnp.concatenate(
        [2 * srcP, 2 * dstP + 1]).reshape(NW, KB2, BE)

    x32 = _padw(x, NPAD, 32)
    ea_pad = jnp.pad(edge_attr, ((0, EPAD - E), (0, 0)))
    ep = _linear(ea_pad, WeT, jnp.zeros((WF,), jnp.float32), 2048)
    ea_mean = _tc_ea_mean(edge_attr)

    zeros32 = jnp.zeros((2, NPAD, 32), jnp.float32)
    z16 = jnp.zeros((NPAD, 16), jnp.float32)
    z32 = jnp.zeros((NPAD, 32), jnp.float32)

    def gat_layer(h32):
        T2v = _linear(h32, Wboth, bboth, 2504)
        T2 = T2v.reshape(2 * NPAD, WF)
        gAll = _sc_gather_all(T2, idxall3)
        exa = _tc_alpha(gAll, ep, attf, M)
        exl = _tc_loop_alpha(T2v, ea_mean, WeT, attf, M)
        denp = _sc_scatter_add(exa, dst3, jnp.stack([exl, z16]))
        den = _tc_den(denp)
        deng = _sc_gather1(den, dst3)
        q = _tc_q(gAll, exa, deng, MT, R)
        qloop = _tc_qloop(T2v, exl, den, MT, R)
        qparts = _sc_scatter_add(q, dst3, jnp.stack([qloop, z32]))
        m1, mW = _tc_m1(qparts, bias32, ggcWT)
        aggm = _sc_gather_scatter_add(mW, src3, dst3, zeros32)
        return _tc_gru(aggm, m1, WihT, bih, WhhT, bhh)

    h1 = gat_layer(x32)
    h2 = gat_layer(h1)

    H = jnp.concatenate([x32[:, :D], h1[:, :D], h2[:, :D]], axis=1)
    H = jnp.pad(H, ((0, 0), (0, 4)))
    hparts = _sc_gather_scatter_add(
        H, src3, dst3, jnp.zeros((2, NPAD, 64), jnp.float32))
    score = _tc_score(hparts, H, WrelT, brel, WrootT)[:, 0:1]

    scoreC = jnp.concatenate(
        [score[:N], jnp.zeros((NP2 - N, 1), jnp.float32)])
    batchC = jnp.concatenate(
        [batch, jnp.full((NP2 - N,), N_GRAPHS, jnp.int32)]).reshape(NP2, 1)
    scoreR = scoreC.reshape(1, NP2)
    batchR = batchC.reshape(1, NP2)
    rank, cnt = _tc_rank(scoreC, scoreR, batchC, batchR)

    H2 = jnp.pad(H[:N], ((0, NP2 - N), (0, 0)))
    g = _tc_pool(H2, scoreC, rank, cnt, batchC)
    out = _tc_mlp(g, W1T, b1, W2T, b2, WoT, bo)
    return out[:, 0]


# speedup vs baseline: 14.9203x; 1.0155x over previous
import functools

import jax
import jax.numpy as jnp
from jax import lax
from jax.experimental import pallas as pl
from jax.experimental.pallas import tpu as pltpu
from jax.experimental.pallas import tpu_sc as plsc

HEADS = 10
D = 20
N_GRAPHS = 64
N = 10000
E = 160000
NPAD = 10016
NW = 32
BE = 80
EPAD = 163840
KB = EPAD // (NW * BE)
KB2 = 2 * KB
NB = 4
WF = 208
NP2 = 10240

_SC_PARAMS = pltpu.CompilerParams(use_tc_tiling_on_sc=False)


def _linear_block(x_ref, w_ref, b_ref, o_ref):
    o_ref[...] = (
        jnp.dot(x_ref[...], w_ref[...], preferred_element_type=jnp.float32)
        + b_ref[...]
    )


def _linear(x, Wt, b, block_rows):
    n, k = x.shape
    m = Wt.shape[1]
    b2 = b.reshape(1, m)
    return pl.pallas_call(
        _linear_block,
        grid=(n // block_rows,),
        in_specs=[
            pl.BlockSpec((block_rows, k), lambda i: (i, 0)),
            pl.BlockSpec((k, m), lambda i: (0, 0)),
            pl.BlockSpec((1, m), lambda i: (0, 0)),
        ],
        out_specs=pl.BlockSpec((block_rows, m), lambda i: (i, 0)),
        out_shape=jax.ShapeDtypeStruct((n, m), jnp.float32),
    )(x, Wt, b2)


def _sc_mesh():
    return plsc.VectorSubcoreMesh(core_axis_name="c", subcore_axis_name="s")


def _sc_gather_all(tbl, idx3):
    W = tbl.shape[1]
    NE = NW * KB2 * BE

    @functools.partial(
        pl.kernel,
        mesh=_sc_mesh(),
        out_type=jax.ShapeDtypeStruct((NE, W), jnp.float32),
        compiler_params=_SC_PARAMS,
        scratch_types=[
            pltpu.VMEM((KB2, BE), jnp.int32),
            pltpu.VMEM((NB * BE, W), jnp.float32),
        ] + [pltpu.SemaphoreType.DMA] * NB,
    )
    def k(t_h, idx_h, o_h, vidx, bufs, *sems):
        cid = lax.axis_index("c")
        sid = lax.axis_index("s")
        wid = sid * 2 + cid
        pltpu.sync_copy(idx_h.at[wid], vidx)

        @pl.loop(0, KB2, step=NB)
        def _(g):
            hs = [pltpu.async_copy(t_h.at[vidx.at[g + b]],
                                   bufs.at[pl.ds(b * BE, BE)], sems[b])
                  for b in range(NB)]
            for b in range(NB):
                hs[b].wait()
            pltpu.sync_copy(
                bufs, o_h.at[pl.ds(wid * (KB2 * BE) + g * BE, NB * BE)])

    return k(tbl, idx3)


def _sc_gather1(tbl, idx3):
    W = tbl.shape[1]
    NB1 = 8

    @functools.partial(
        pl.kernel,
        mesh=_sc_mesh(),
        out_type=jax.ShapeDtypeStruct((EPAD, W), jnp.float32),
        compiler_params=_SC_PARAMS,
        scratch_types=[
            pltpu.VMEM((KB, BE), jnp.int32),
            pltpu.VMEM((NB1 * BE, W), jnp.float32),
        ] + [pltpu.SemaphoreType.DMA] * NB1,
    )
    def k(t_h, idx_h, o_h, vidx, bufs, *sems):
        cid = lax.axis_index("c")
        sid = lax.axis_index("s")
        wid = sid * 2 + cid
        pltpu.sync_copy(idx_h.at[wid], vidx)

        @pl.loop(0, KB, step=NB1)
        def _(g):
            hs = [pltpu.async_copy(t_h.at[vidx.at[g + b]],
                                   bufs.at[pl.ds(b * BE, BE)], sems[b])
                  for b in range(NB1)]
            for b in range(NB1):
                hs[b].wait()
            pltpu.sync_copy(
                bufs, o_h.at[pl.ds(wid * (KB * BE) + g * BE, NB1 * BE)])

    return k(tbl, idx3)


def _sc_scatter_add(vals, dst3, init2):
    W = vals.shape[1]
    NB1 = 8

    @functools.partial(
        pl.kernel,
        mesh=_sc_mesh(),
        out_type=jax.ShapeDtypeStruct((2, NPAD, W), jnp.float32),
        compiler_params=_SC_PARAMS,
        scratch_types=[
            pltpu.VMEM((KB, BE), jnp.int32),
            pltpu.VMEM((NB1 * BE, W), jnp.float32),
            pltpu.VMEM_SHARED((NPAD, W), jnp.float32),
        ] + [pltpu.SemaphoreType.DMA] * (NB1 + 1),
    )
    def k(v_h, dst_h, init_h, out_h, didx, bufs, acc, *sems):
        cid = lax.axis_index("c")
        sid = lax.axis_index("s")
        wid = sid * 2 + cid
        rsem = sems[NB1]

        @pl.when(sid == 0)
        def _():
            pltpu.sync_copy(init_h.at[cid], acc)

        plsc.subcore_barrier()
        pltpu.sync_copy(dst_h.at[wid], didx)

        @pl.loop(0, KB, step=NB1)
        def _(g):
            rd = pltpu.async_copy(
                v_h.at[pl.ds(wid * (KB * BE) + g * BE, NB1 * BE)], bufs, rsem)
            rd.wait()
            ss = [pltpu.async_copy(bufs.at[pl.ds(b * BE, BE)],
                                   acc.at[didx.at[g + b]], sems[b], add=True)
                  for b in range(NB1)]
            for b in range(NB1):
                ss[b].wait()

        plsc.subcore_barrier()

        @pl.when(sid == 0)
        def _():
            pltpu.sync_copy(acc, out_h.at[cid])

    return k(vals, dst3, init2)


def _sc_gather_scatter_add(table, src3, dst3, init2):
    W = table.shape[1]
    NB1 = 8

    @functools.partial(
        pl.kernel,
        mesh=_sc_mesh(),
        out_type=jax.ShapeDtypeStruct((2, NPAD, W), jnp.float32),
        compiler_params=_SC_PARAMS,
        scratch_types=[
            pltpu.VMEM((KB, BE), jnp.int32),
            pltpu.VMEM((KB, BE), jnp.int32),
            pltpu.VMEM((NB1, BE, W), jnp.float32),
            pltpu.VMEM_SHARED((NPAD, W), jnp.float32),
        ] + [pltpu.SemaphoreType.DMA] * (2 * NB1),
    )
    def k(tbl_h, src_h, dst_h, init_h, out_h, sidx, didx, bufs, acc, *sems):
        cid = lax.axis_index("c")
        sid = lax.axis_index("s")
        wid = sid * 2 + cid

        @pl.when(sid == 0)
        def _():
            pltpu.sync_copy(init_h.at[cid], acc)

        plsc.subcore_barrier()
        pltpu.sync_copy(src_h.at[wid], sidx)
        pltpu.sync_copy(dst_h.at[wid], didx)

        @pl.loop(0, KB, step=NB1)
        def _(g):
            hs = [pltpu.async_copy(tbl_h.at[sidx.at[g + b]], bufs.at[b],
                                   sems[b]) for b in range(NB1)]
            ss = []
            for b in range(NB1):
                hs[b].wait()
                ss.append(pltpu.async_copy(bufs.at[b], acc.at[didx.at[g + b]],
                                           sems[NB1 + b], add=True))
            for b in range(NB1):
                ss[b].wait()

        plsc.subcore_barrier()

        @pl.when(sid == 0)
        def _():
            pltpu.sync_copy(acc, out_h.at[cid])

    return k(table, src3, dst3, init2)


def _tc_alpha(gAll, ep, attf, M):
    def body(xl_ref, xr_ref, ep_ref, at_ref, m_ref, o_ref):
        z = xl_ref[...] + xr_ref[...] + ep_ref[...]
        m = jnp.where(z >= 0, z, 0.2 * z)
        s = m * at_ref[...]
        alpha = jnp.dot(s, m_ref[...], preferred_element_type=jnp.float32)
        o_ref[...] = jnp.exp(alpha)

    B = 2048
    nblk = EPAD // B
    return pl.pallas_call(
        body,
        grid=(nblk,),
        in_specs=[
            pl.BlockSpec((B, WF), lambda i: (i, 0)),
            pl.BlockSpec((B, WF), lambda i: (i + nblk, 0)),
            pl.BlockSpec((B, WF), lambda i: (i, 0)),
            pl.BlockSpec((1, WF), lambda i: (0, 0)),
            pl.BlockSpec((WF, 16), lambda i: (0, 0)),
        ],
        out_specs=pl.BlockSpec((B, 16), lambda i: (i, 0)),
        out_shape=jax.ShapeDtypeStruct((EPAD, 16), jnp.float32),
    )(gAll, gAll, ep, attf, M)


def _tc_loop_alpha(T2v, ea_mean, WeT, attf, M):
    def body(t_ref, eam_ref, we_ref, at_ref, m_ref, o_ref):
        epm = jnp.dot(eam_ref[...], we_ref[...],
                      preferred_element_type=jnp.float32)
        z = t_ref[:, :WF] + t_ref[:, WF:] + epm
        m = jnp.where(z >= 0, z, 0.2 * z)
        s = m * at_ref[...]
        o_ref[...] = jnp.exp(
            jnp.dot(s, m_ref[...], preferred_element_type=jnp.float32))

    B = 2504
    return pl.pallas_call(
        body,
        grid=(NPAD // B,),
        in_specs=[
            pl.BlockSpec((B, 2 * WF), lambda i: (i, 0)),
            pl.BlockSpec((1, 8), lambda i: (0, 0)),
            pl.BlockSpec((8, WF), lambda i: (0, 0)),
            pl.BlockSpec((1, WF), lambda i: (0, 0)),
            pl.BlockSpec((WF, 16), lambda i: (0, 0)),
        ],
        out_specs=pl.BlockSpec((B, 16), lambda i: (i, 0)),
        out_shape=jax.ShapeDtypeStruct((NPAD, 16), jnp.float32),
    )(T2v, ea_mean, WeT, attf, M)


def _tc_ea_mean(ea):
    def body(ea_ref, o_ref):
        @pl.when(pl.program_id(0) == 0)
        def _():
            o_ref[...] = jnp.zeros_like(o_ref)
        o_ref[...] += jnp.sum(ea_ref[...], axis=0, keepdims=True) / E

    B = 2000
    return pl.pallas_call(
        body,
        grid=(E // B,),
        in_specs=[pl.BlockSpec((B, 8), lambda i: (i, 0))],
        out_specs=pl.BlockSpec((1, 8), lambda i: (0, 0)),
        out_shape=jax.ShapeDtypeStruct((1, 8), jnp.float32),
    )(ea)


def _tc_den(parts):
    def body(p_ref, o_ref):
        o_ref[...] = p_ref[0] + p_ref[1]

    return pl.pallas_call(
        body,
        in_specs=[pl.BlockSpec((2, NPAD, 16), lambda: (0, 0, 0))],
        out_specs=pl.BlockSpec((NPAD, 16), lambda: (0, 0)),
        out_shape=jax.ShapeDtypeStruct((NPAD, 16), jnp.float32),
    )(parts)


def _tc_q(xlg, exa, deng, MT, R):
    def body(xl_ref, ex_ref, dn_ref, mt_ref, r_ref, o_ref):
        a = ex_ref[...] / (dn_ref[...] + 1e-16)
        arep = jnp.dot(a, mt_ref[...], preferred_element_type=jnp.float32)
        w = arep * xl_ref[...]
        o_ref[...] = jnp.dot(w, r_ref[...], preferred_element_type=jnp.float32)

    B = 2048
    return pl.pallas_call(
        body,
        grid=(EPAD // B,),
        in_specs=[
            pl.BlockSpec((B, WF), lambda i: (i, 0)),
            pl.BlockSpec((B, 16), lambda i: (i, 0)),
            pl.BlockSpec((B, 16), lambda i: (i, 0)),
            pl.BlockSpec((16, WF), lambda i: (0, 0)),
            pl.BlockSpec((WF, 32), lambda i: (0, 0)),
        ],
        out_specs=pl.BlockSpec((B, 32), lambda i: (i, 0)),
        out_shape=jax.ShapeDtypeStruct((EPAD, 32), jnp.float32),
    )(xlg, exa, deng, MT, R)


def _tc_qloop(xl, exl, den, MT, R):
    def body(xl_ref, ex_ref, dn_ref, mt_ref, r_ref, o_ref):
        a = ex_ref[...] / (dn_ref[...] + 1e-16)
        arep = jnp.dot(a, mt_ref[...], preferred_element_type=jnp.float32)
        w = arep * xl_ref[:, :WF]
        o_ref[...] = jnp.dot(w, r_ref[...], preferred_element_type=jnp.float32)

    B = 2504
    return pl.pallas_call(
        body,
        grid=(NPAD // B,),
        in_specs=[
            pl.BlockSpec((B, 2 * WF), lambda i: (i, 0)),
            pl.BlockSpec((B, 16), lambda i: (i, 0)),
            pl.BlockSpec((B, 16), lambda i: (i, 0)),
            pl.BlockSpec((16, WF), lambda i: (0, 0)),
            pl.BlockSpec((WF, 32), lambda i: (0, 0)),
        ],
        out_specs=pl.BlockSpec((B, 32), lambda i: (i, 0)),
        out_shape=jax.ShapeDtypeStruct((NPAD, 32), jnp.float32),
    )(xl, exl, den, MT, R)


def _tc_m1(qparts, bias32, ggcWT):
    def body(q_ref, b_ref, w_ref, m_ref, mw_ref):
        m1 = (q_ref[0] + q_ref[1]) * (1.0 / HEADS) + b_ref[...]
        m_ref[...] = m1
        mw_ref[...] = jnp.dot(m1, w_ref[...],
                              preferred_element_type=jnp.float32)

    return pl.pallas_call(
        body,
        in_specs=[
            pl.BlockSpec((2, NPAD, 32), lambda: (0, 0, 0)),
            pl.BlockSpec((1, 32), lambda: (0, 0)),
            pl.BlockSpec((32, 32), lambda: (0, 0)),
        ],
        out_specs=[
            pl.BlockSpec((NPAD, 32), lambda: (0, 0)),
            pl.BlockSpec((NPAD, 32), lambda: (0, 0)),
        ],
        out_shape=[
            jax.ShapeDtypeStruct((NPAD, 32), jnp.float32),
            jax.ShapeDtypeStruct((NPAD, 32), jnp.float32),
        ],
    )(qparts, bias32, ggcWT)


def _tc_gru(mparts, m1, WihT, bih, WhhT, bhh):
    def body(p_ref, x_ref, wi_ref, bi_ref, wh_ref, bh_ref, o_ref):
        agg = p_ref[0] + p_ref[1]
        x = x_ref[...]
        gi = jnp.dot(agg, wi_ref[...],
                     preferred_element_type=jnp.float32) + bi_ref[...]
        gh = jnp.dot(x, wh_ref[...],
                     preferred_element_type=jnp.float32) + bh_ref[...]
        r = jax.nn.sigmoid(gi[:, 0:D] + gh[:, 0:D])
        z = jax.nn.sigmoid(gi[:, D:2 * D] + gh[:, D:2 * D])
        nt = jnp.tanh(gi[:, 2 * D:3 * D] + r * gh[:, 2 * D:3 * D])
        hn = (1.0 - z) * nt + z * x[:, 0:D]
        o_ref[...] = jnp.pad(hn, ((0, 0), (0, 12)))

    return pl.pallas_call(
        body,
        in_specs=[
            pl.BlockSpec((2, NPAD, 32), lambda: (0, 0, 0)),
            pl.BlockSpec((NPAD, 32), lambda: (0, 0)),
            pl.BlockSpec((32, 64), lambda: (0, 0)),
            pl.BlockSpec((1, 64), lambda: (0, 0)),
            pl.BlockSpec((32, 64), lambda: (0, 0)),
            pl.BlockSpec((1, 64), lambda: (0, 0)),
        ],
        out_specs=pl.BlockSpec((NPAD, 32), lambda: (0, 0)),
        out_shape=jax.ShapeDtypeStruct((NPAD, 32), jnp.float32),
    )(mparts, m1, WihT, bih, WhhT, bhh)


def _tc_score(hparts, H, WrelT, brel, WrootT):
    def body(p_ref, h_ref, wr_ref, br_ref, wo_ref, o_ref):
        agg = p_ref[0] + p_ref[1]
        o_ref[...] = (
            jnp.dot(agg, wr_ref[...], preferred_element_type=jnp.float32)
            + br_ref[...]
            + jnp.dot(h_ref[...], wo_ref[...],
                      preferred_element_type=jnp.float32)
        )

    return pl.pallas_call(
        body,
        in_specs=[
            pl.BlockSpec((2, NPAD, 64), lambda: (0, 0, 0)),
            pl.BlockSpec((NPAD, 64), lambda: (0, 0)),
            pl.BlockSpec((64, 8), lambda: (0, 0)),
            pl.BlockSpec((1, 8), lambda: (0, 0)),
            pl.BlockSpec((64, 8), lambda: (0, 0)),
        ],
        out_specs=pl.BlockSpec((NPAD, 8), lambda: (0, 0)),
        out_shape=jax.ShapeDtypeStruct((NPAD, 8), jnp.float32),
    )(hparts, H, WrelT, brel, WrootT)


def _tc_rank(scoreC, scoreR, batchC, batchR):
    BI, BJ = 512, 2048

    def body(si_ref, sj_ref, bi_ref, bj_ref, r_ref, c_ref):
        i0 = pl.program_id(0) * BI
        j0 = pl.program_id(1) * BJ

        @pl.when(pl.program_id(1) == 0)
        def _():
            r_ref[...] = jnp.zeros_like(r_ref)
            c_ref[...] = jnp.zeros_like(c_ref)

        ii = i0 + lax.broadcasted_iota(jnp.int32, (BI, BJ), 0)
        jj = j0 + lax.broadcasted_iota(jnp.int32, (BI, BJ), 1)
        eq = bi_ref[...] == bj_ref[...]
        sj = sj_ref[...]
        si = si_ref[...]
        ahead = (sj > si) | ((sj == si) & (jj < ii))
        contrib = jnp.where(eq & ahead, 1.0, 0.0)
        cgrp = jnp.where(eq, 1.0, 0.0)
        r_ref[...] += jnp.sum(contrib, axis=1, keepdims=True)
        c_ref[...] += jnp.sum(cgrp, axis=1, keepdims=True)

    return pl.pallas_call(
        body,
        grid=(NP2 // BI, NP2 // BJ),
        in_specs=[
            pl.BlockSpec((BI, 1), lambda i, j: (i, 0)),
            pl.BlockSpec((1, BJ), lambda i, j: (0, j)),
            pl.BlockSpec((BI, 1), lambda i, j: (i, 0)),
            pl.BlockSpec((1, BJ), lambda i, j: (0, j)),
        ],
        out_specs=[
            pl.BlockSpec((BI, 1), lambda i, j: (i, 0)),
            pl.BlockSpec((BI, 1), lambda i, j: (i, 0)),
        ],
        out_shape=[
            jax.ShapeDtypeStruct((NP2, 1), jnp.float32),
            jax.ShapeDtypeStruct((NP2, 1), jnp.float32),
        ],
    )(scoreC, scoreR, batchC, batchR)


def _tc_pool(H, score, rank, cnt, batchC):
    B = 1024

    def body(h_ref, s_ref, r_ref, c_ref, b_ref, o_ref):
        @pl.when(pl.program_id(0) == 0)
        def _():
            o_ref[...] = jnp.zeros_like(o_ref)

        kq = jnp.ceil(0.3 * c_ref[...])
        mask = jnp.where(r_ref[...] < kq, 1.0, 0.0)
        hs = h_ref[...] * jnp.tanh(s_ref[...]) * mask
        gid = lax.broadcasted_iota(jnp.int32, (B, 64), 1)
        oh = jnp.where(b_ref[...] == gid, 1.0, 0.0)
        o_ref[...] += lax.dot_general(
            oh, hs, (((0,), (0,)), ((), ())),
            preferred_element_type=jnp.float32)

    return pl.pallas_call(
        body,
        grid=(NP2 // B,),
        in_specs=[
            pl.BlockSpec((B, 64), lambda i: (i, 0)),
            pl.BlockSpec((B, 1), lambda i: (i, 0)),
            pl.BlockSpec((B, 1), lambda i: (i, 0)),
            pl.BlockSpec((B, 1), lambda i: (i, 0)),
            pl.BlockSpec((B, 1), lambda i: (i, 0)),
        ],
        out_specs=pl.BlockSpec((64, 64), lambda i: (0, 0)),
        out_shape=jax.ShapeDtypeStruct((64, 64), jnp.float32),
    )(H, score, rank, cnt, batchC)


def _tc_mlp(g, W1T, b1, W2T, b2, WoT, bo):
    def body(g_ref, w1, b1r, w2, b2r, wo, bor, o_ref):
        a = jnp.dot(g_ref[...], w1[...],
                    preferred_element_type=jnp.float32) + b1r[...]
        a = jnp.where(a >= 0, a, 0.01 * a)
        a = jnp.dot(a, w2[...], preferred_element_type=jnp.float32) + b2r[...]
        a = jnp.where(a >= 0, a, 0.01 * a)
        o_ref[...] = jnp.dot(a, wo[...],
                             preferred_element_type=jnp.float32) + bor[...]

    return pl.pallas_call(
        body,
        in_specs=[
            pl.BlockSpec((64, 64), lambda: (0, 0)),
            pl.BlockSpec((64, 64), lambda: (0, 0)),
            pl.BlockSpec((1, 64), lambda: (0, 0)),
            pl.BlockSpec((64, 32), lambda: (0, 0)),
            pl.BlockSpec((1, 32), lambda: (0, 0)),
            pl.BlockSpec((32, 8), lambda: (0, 0)),
            pl.BlockSpec((1, 8), lambda: (0, 0)),
        ],
        out_specs=pl.BlockSpec((64, 8), lambda: (0, 0)),
        out_shape=jax.ShapeDtypeStruct((64, 8), jnp.float32),
    )(g, W1T, b1, W2T, b2, WoT, bo)


def _padw(a, rows, cols):
    return jnp.pad(a, ((0, rows - a.shape[0]), (0, cols - a.shape[1])))


def kernel(x, edge_index, edge_attr, batch, params):
    p = params
    f = jnp.arange(WF)
    valid = (f < HEADS * D)
    M = ((f[:, None] // D == jnp.arange(16)[None, :]) &
         valid[:, None]).astype(jnp.float32)
    MT = M.T
    R = ((f[:, None] % D == jnp.arange(32)[None, :]) &
         valid[:, None]).astype(jnp.float32)
    attf = jnp.pad(p['gat_att'].reshape(1, HEADS * D), ((0, 0), (0, 8)))

    Wboth = jnp.concatenate(
        [_padw(p['gat_Wl'].T, 32, WF), _padw(p['gat_Wr'].T, 32, WF)], axis=1)
    bboth = jnp.concatenate(
        [jnp.pad(p['gat_bl'], (0, 8)), jnp.pad(p['gat_br'], (0, 8))])
    WeT = jnp.pad(p['gat_We'].T, ((0, 0), (0, 8)))
    bias32 = jnp.pad(p['gat_bias'], (0, 12)).reshape(1, 32)
    ggcWT = _padw(p['ggc_weight'], 32, 32)
    WihT = _padw(p['gru_Wih'].T, 32, 64)
    bih = jnp.pad(p['gru_bih'], (0, 4)).reshape(1, 64)
    WhhT = _padw(p['gru_Whh'].T, 32, 64)
    bhh = jnp.pad(p['gru_bhh'], (0, 4)).reshape(1, 64)
    WrelT = _padw(p['pool_Wrel'].T, 64, 8)
    brel = jnp.pad(p['pool_brel'], (0, 7)).reshape(1, 8)
    WrootT = _padw(p['pool_Wroot'].T, 64, 8)
    W1T = _padw(p['fc1_W'].T, 64, 64)
    b1 = jnp.pad(p['fc1_b'], (0, 24)).reshape(1, 64)
    W2T = _padw(p['fc2_W'].T, 64, 32)
    b2 = jnp.pad(p['fc2_b'], (0, 2)).reshape(1, 32)
    WoT = _padw(p['out_W'].T, 32, 8)
    bo = jnp.pad(p['out_b'], (0, 7)).reshape(1, 8)

    srcP = jnp.concatenate(
        [edge_index[0], jnp.zeros((EPAD - E,), jnp.int32)])
    dstP = jnp.concatenate(
        [edge_index[1], jnp.full((EPAD - E,), N, jnp.int32)])
    src3 = srcP.reshape(NW, KB, BE)
    dst3 = dstP.reshape(NW, KB, BE)
    idxall3 = jnp.concatenate(
        [2 * srcP, 2 * dstP + 1]).reshape(NW, KB2, BE)

    x32 = _padw(x, NPAD, 32)
    ea_pad = jnp.pad(edge_attr, ((0, EPAD - E), (0, 0)))
    ep = _linear(ea_pad, WeT, jnp.zeros((WF,), jnp.float32), 2048)
    ea_mean = _tc_ea_mean(edge_attr)

    zeros32 = jnp.zeros((2, NPAD, 32), jnp.float32)
    z16 = jnp.zeros((NPAD, 16), jnp.float32)
    z32 = jnp.zeros((NPAD, 32), jnp.float32)

    def gat_layer(h32):
        T2v = _linear(h32, Wboth, bboth, 2504)
        T2 = T2v.reshape(2 * NPAD, WF)
        gAll = _sc_gather_all(T2, idxall3)
        exa = _tc_alpha(gAll, ep, attf, M)
        exl = _tc_loop_alpha(T2v, ea_mean, WeT, attf, M)
        denp = _sc_scatter_add(exa, dst3, jnp.stack([exl, z16]))
        den = _tc_den(denp)
        deng = _sc_gather1(den, dst3)
        q = _tc_q(gAll, exa, deng, MT, R)
        qloop = _tc_qloop(T2v, exl, den, MT, R)
        qparts = _sc_scatter_add(q, dst3, jnp.stack([qloop, z32]))
        m1, mW = _tc_m1(qparts, bias32, ggcWT)
        aggm = _sc_gather_scatter_add(mW, src3, dst3, zeros32)
        return _tc_gru(aggm, m1, WihT, bih, WhhT, bhh)

    h1 = gat_layer(x32)
    h2 = gat_layer(h1)

    H = jnp.concatenate([x32[:, :D], h1[:, :D], h2[:, :D]], axis=1)
    H = jnp.pad(H, ((0, 0), (0, 4)))
    hparts = _sc_gather_scatter_add(
        H, src3, dst3, jnp.zeros((2, NPAD, 64), jnp.float32))
    score = _tc_score(hparts, H, WrelT, brel, WrootT)[:, 0:1]

    scoreC = jnp.concatenate(
        [score[:N], jnp.zeros((NP2 - N, 1), jnp.float32)])
    batchC = jnp.concatenate(
        [batch, jnp.full((NP2 - N,), N_GRAPHS, jnp.int32)]).reshape(NP2, 1)
    scoreR = scoreC.reshape(1, NP2)
    batchR = batchC.reshape(1, NP2)
    rank, cnt = _tc_rank(scoreC, scoreR, batchC, batchR)

    H2 = jnp.pad(H[:N], ((0, NP2 - N), (0, 0)))
    g = _tc_pool(H2, scoreC, rank, cnt, batchC)
    out = _tc_mlp(g, W1T, b1, W2T, b2, WoT, bo)
    return out[:, 0]


# speedup vs baseline: 15.3450x; 1.0285x over previous
import functools

import jax
import jax.numpy as jnp
from jax import lax
from jax.experimental import pallas as pl
from jax.experimental.pallas import tpu as pltpu
from jax.experimental.pallas import tpu_sc as plsc

HEADS = 10
D = 20
N_GRAPHS = 64
N = 10000
E = 160000
NPAD = 10016
NW = 32
BE = 80
EPAD = 163840
KB = EPAD // (NW * BE)
KB2 = 2 * KB
NB = 4
WF = 208
NP2 = 10240

_SC_PARAMS = pltpu.CompilerParams(use_tc_tiling_on_sc=False)


def _linear_block(x_ref, w_ref, b_ref, o_ref):
    o_ref[...] = (
        jnp.dot(x_ref[...], w_ref[...], preferred_element_type=jnp.float32)
        + b_ref[...]
    )


def _linear(x, Wt, b, block_rows):
    n, k = x.shape
    m = Wt.shape[1]
    b2 = b.reshape(1, m)
    return pl.pallas_call(
        _linear_block,
        grid=(n // block_rows,),
        in_specs=[
            pl.BlockSpec((block_rows, k), lambda i: (i, 0)),
            pl.BlockSpec((k, m), lambda i: (0, 0)),
            pl.BlockSpec((1, m), lambda i: (0, 0)),
        ],
        out_specs=pl.BlockSpec((block_rows, m), lambda i: (i, 0)),
        out_shape=jax.ShapeDtypeStruct((n, m), jnp.float32),
    )(x, Wt, b2)


def _sc_mesh():
    return plsc.VectorSubcoreMesh(core_axis_name="c", subcore_axis_name="s")


def _sc_gather_all(tbl, idx3):
    W = tbl.shape[1]
    NE = NW * KB2 * BE

    @functools.partial(
        pl.kernel,
        mesh=_sc_mesh(),
        out_type=jax.ShapeDtypeStruct((NE, W), jnp.float32),
        compiler_params=_SC_PARAMS,
        scratch_types=[
            pltpu.VMEM((KB2, BE), jnp.int32),
            pltpu.VMEM((NB * BE, W), jnp.float32),
        ] + [pltpu.SemaphoreType.DMA] * NB,
    )
    def k(t_h, idx_h, o_h, vidx, bufs, *sems):
        cid = lax.axis_index("c")
        sid = lax.axis_index("s")
        wid = sid * 2 + cid
        pltpu.sync_copy(idx_h.at[wid], vidx)

        @pl.loop(0, KB2, step=NB)
        def _(g):
            hs = [pltpu.async_copy(t_h.at[vidx.at[g + b]],
                                   bufs.at[pl.ds(b * BE, BE)], sems[b])
                  for b in range(NB)]
            for b in range(NB):
                hs[b].wait()
            pltpu.sync_copy(
                bufs, o_h.at[pl.ds(wid * (KB2 * BE) + g * BE, NB * BE)])

    return k(tbl, idx3)


def _sc_scatter_add(vals, dst3, init2):
    W = vals.shape[1]
    NB1 = 4 if W > 64 else 8

    @functools.partial(
        pl.kernel,
        mesh=_sc_mesh(),
        out_type=jax.ShapeDtypeStruct((2, NPAD, W), jnp.float32),
        compiler_params=_SC_PARAMS,
        scratch_types=[
            pltpu.VMEM((KB, BE), jnp.int32),
            pltpu.VMEM((NB1 * BE, W), jnp.float32),
            pltpu.VMEM_SHARED((NPAD, W), jnp.float32),
        ] + [pltpu.SemaphoreType.DMA] * (NB1 + 1),
    )
    def k(v_h, dst_h, init_h, out_h, didx, bufs, acc, *sems):
        cid = lax.axis_index("c")
        sid = lax.axis_index("s")
        wid = sid * 2 + cid
        rsem = sems[NB1]

        @pl.when(sid == 0)
        def _():
            pltpu.sync_copy(init_h.at[cid], acc)

        plsc.subcore_barrier()
        pltpu.sync_copy(dst_h.at[wid], didx)

        @pl.loop(0, KB, step=NB1)
        def _(g):
            rd = pltpu.async_copy(
                v_h.at[pl.ds(wid * (KB * BE) + g * BE, NB1 * BE)], bufs, rsem)
            rd.wait()
            ss = [pltpu.async_copy(bufs.at[pl.ds(b * BE, BE)],
                                   acc.at[didx.at[g + b]], sems[b], add=True)
                  for b in range(NB1)]
            for b in range(NB1):
                ss[b].wait()

        plsc.subcore_barrier()

        @pl.when(sid == 0)
        def _():
            pltpu.sync_copy(acc, out_h.at[cid])

    return k(vals, dst3, init2)


def _sc_gather_scatter_add(table, src3, dst3, init2):
    W = table.shape[1]
    NB1 = 8

    @functools.partial(
        pl.kernel,
        mesh=_sc_mesh(),
        out_type=jax.ShapeDtypeStruct((2, NPAD, W), jnp.float32),
        compiler_params=_SC_PARAMS,
        scratch_types=[
            pltpu.VMEM((KB, BE), jnp.int32),
            pltpu.VMEM((KB, BE), jnp.int32),
            pltpu.VMEM((NB1, BE, W), jnp.float32),
            pltpu.VMEM_SHARED((NPAD, W), jnp.float32),
        ] + [pltpu.SemaphoreType.DMA] * (2 * NB1),
    )
    def k(tbl_h, src_h, dst_h, init_h, out_h, sidx, didx, bufs, acc, *sems):
        cid = lax.axis_index("c")
        sid = lax.axis_index("s")
        wid = sid * 2 + cid

        @pl.when(sid == 0)
        def _():
            pltpu.sync_copy(init_h.at[cid], acc)

        plsc.subcore_barrier()
        pltpu.sync_copy(src_h.at[wid], sidx)
        pltpu.sync_copy(dst_h.at[wid], didx)

        @pl.loop(0, KB, step=NB1)
        def _(g):
            hs = [pltpu.async_copy(tbl_h.at[sidx.at[g + b]], bufs.at[b],
                                   sems[b]) for b in range(NB1)]
            ss = []
            for b in range(NB1):
                hs[b].wait()
                ss.append(pltpu.async_copy(bufs.at[b], acc.at[didx.at[g + b]],
                                           sems[NB1 + b], add=True))
            for b in range(NB1):
                ss[b].wait()

        plsc.subcore_barrier()

        @pl.when(sid == 0)
        def _():
            pltpu.sync_copy(acc, out_h.at[cid])

    return k(table, src3, dst3, init2)


def _tc_alpha(gAll, eaP, WeT, attf, M):
    def body(xl_ref, xr_ref, ea_ref, we_ref, at_ref, m_ref, ex_ref):
        ep = jnp.dot(ea_ref[...], we_ref[...],
                     preferred_element_type=jnp.float32)
        z = xl_ref[...] + xr_ref[...] + ep
        m = jnp.where(z >= 0, z, 0.2 * z)
        s = m * at_ref[...]
        alpha = jnp.dot(s, m_ref[...], preferred_element_type=jnp.float32)
        ex_ref[...] = jnp.exp(alpha)

    B = 2048
    nblk = EPAD // B
    return pl.pallas_call(
        body,
        grid=(nblk,),
        in_specs=[
            pl.BlockSpec((B, WF), lambda i: (i, 0)),
            pl.BlockSpec((B, WF), lambda i: (i + nblk, 0)),
            pl.BlockSpec((B, 8), lambda i: (i, 0)),
            pl.BlockSpec((8, WF), lambda i: (0, 0)),
            pl.BlockSpec((1, WF), lambda i: (0, 0)),
            pl.BlockSpec((WF, 16), lambda i: (0, 0)),
        ],
        out_specs=pl.BlockSpec((B, 16), lambda i: (i, 0)),
        out_shape=jax.ShapeDtypeStruct((EPAD, 16), jnp.float32),
    )(gAll, gAll, eaP, WeT, attf, M)


def _sc_gather1(tbl, idx3):
    W = tbl.shape[1]
    NB1 = 8

    @functools.partial(
        pl.kernel,
        mesh=_sc_mesh(),
        out_type=jax.ShapeDtypeStruct((EPAD, W), jnp.float32),
        compiler_params=_SC_PARAMS,
        scratch_types=[
            pltpu.VMEM((KB, BE), jnp.int32),
            pltpu.VMEM((NB1 * BE, W), jnp.float32),
        ] + [pltpu.SemaphoreType.DMA] * NB1,
    )
    def k(t_h, idx_h, o_h, vidx, bufs, *sems):
        cid = lax.axis_index("c")
        sid = lax.axis_index("s")
        wid = sid * 2 + cid
        pltpu.sync_copy(idx_h.at[wid], vidx)

        @pl.loop(0, KB, step=NB1)
        def _(g):
            hs = [pltpu.async_copy(t_h.at[vidx.at[g + b]],
                                   bufs.at[pl.ds(b * BE, BE)], sems[b])
                  for b in range(NB1)]
            for b in range(NB1):
                hs[b].wait()
            pltpu.sync_copy(
                bufs, o_h.at[pl.ds(wid * (KB * BE) + g * BE, NB1 * BE)])

    return k(tbl, idx3)


def _tc_loop_alpha(T2v, ea_mean, WeT, attf, M, MT):
    def body(t_ref, eam_ref, we_ref, at_ref, m_ref, ex_ref):
        epm = jnp.dot(eam_ref[...], we_ref[...],
                      preferred_element_type=jnp.float32)
        z = t_ref[:, :WF] + t_ref[:, WF:] + epm
        m = jnp.where(z >= 0, z, 0.2 * z)
        s = m * at_ref[...]
        ex_ref[...] = jnp.exp(
            jnp.dot(s, m_ref[...], preferred_element_type=jnp.float32))

    B = 2504
    return pl.pallas_call(
        body,
        grid=(NPAD // B,),
        in_specs=[
            pl.BlockSpec((B, 2 * WF), lambda i: (i, 0)),
            pl.BlockSpec((1, 8), lambda i: (0, 0)),
            pl.BlockSpec((8, WF), lambda i: (0, 0)),
            pl.BlockSpec((1, WF), lambda i: (0, 0)),
            pl.BlockSpec((WF, 16), lambda i: (0, 0)),
        ],
        out_specs=pl.BlockSpec((B, 16), lambda i: (i, 0)),
        out_shape=jax.ShapeDtypeStruct((NPAD, 16), jnp.float32),
    )(T2v, ea_mean, WeT, attf, M)


def _tc_ea_mean(ea):
    def body(ea_ref, o_ref):
        @pl.when(pl.program_id(0) == 0)
        def _():
            o_ref[...] = jnp.zeros_like(o_ref)
        o_ref[...] += jnp.sum(ea_ref[...], axis=0, keepdims=True) / E

    B = 2000
    return pl.pallas_call(
        body,
        grid=(E // B,),
        in_specs=[pl.BlockSpec((B, 8), lambda i: (i, 0))],
        out_specs=pl.BlockSpec((1, 8), lambda i: (0, 0)),
        out_shape=jax.ShapeDtypeStruct((1, 8), jnp.float32),
    )(ea)


def _tc_den(parts):
    def body(p_ref, o_ref):
        o_ref[...] = p_ref[0] + p_ref[1]

    return pl.pallas_call(
        body,
        in_specs=[pl.BlockSpec((2, NPAD, 16), lambda: (0, 0, 0))],
        out_specs=pl.BlockSpec((NPAD, 16), lambda: (0, 0)),
        out_shape=jax.ShapeDtypeStruct((NPAD, 16), jnp.float32),
    )(parts)


def _tc_q(gAll, exa, deng, MT, R):
    def body(xl_ref, ex_ref, dn_ref, mt_ref, r_ref, o_ref):
        a = ex_ref[...] / (dn_ref[...] + 1e-16)
        arep = jnp.dot(a, mt_ref[...], preferred_element_type=jnp.float32)
        w = arep * xl_ref[...]
        o_ref[...] = jnp.dot(w, r_ref[...], preferred_element_type=jnp.float32)

    B = 2048
    return pl.pallas_call(
        body,
        grid=(EPAD // B,),
        in_specs=[
            pl.BlockSpec((B, WF), lambda i: (i, 0)),
            pl.BlockSpec((B, 16), lambda i: (i, 0)),
            pl.BlockSpec((B, 16), lambda i: (i, 0)),
            pl.BlockSpec((16, WF), lambda i: (0, 0)),
            pl.BlockSpec((WF, 32), lambda i: (0, 0)),
        ],
        out_specs=pl.BlockSpec((B, 32), lambda i: (i, 0)),
        out_shape=jax.ShapeDtypeStruct((EPAD, 32), jnp.float32),
    )(gAll, exa, deng, MT, R)


def _tc_qloop(T2v, exl, den, MT, R):
    def body(xl_ref, ex_ref, dn_ref, mt_ref, r_ref, o_ref):
        a = ex_ref[...] / (dn_ref[...] + 1e-16)
        arep = jnp.dot(a, mt_ref[...], preferred_element_type=jnp.float32)
        w = arep * xl_ref[:, :WF]
        o_ref[...] = jnp.dot(w, r_ref[...], preferred_element_type=jnp.float32)

    B = 2504
    return pl.pallas_call(
        body,
        grid=(NPAD // B,),
        in_specs=[
            pl.BlockSpec((B, 2 * WF), lambda i: (i, 0)),
            pl.BlockSpec((B, 16), lambda i: (i, 0)),
            pl.BlockSpec((B, 16), lambda i: (i, 0)),
            pl.BlockSpec((16, WF), lambda i: (0, 0)),
            pl.BlockSpec((WF, 32), lambda i: (0, 0)),
        ],
        out_specs=pl.BlockSpec((B, 32), lambda i: (i, 0)),
        out_shape=jax.ShapeDtypeStruct((NPAD, 32), jnp.float32),
    )(T2v, exl, den, MT, R)


def _tc_m1(qparts, bias32, ggcWT):
    def body(q_ref, b_ref, w_ref, m_ref, mw_ref):
        m1 = (q_ref[0] + q_ref[1]) * (1.0 / HEADS) + b_ref[...]
        m_ref[...] = m1
        mw_ref[...] = jnp.dot(m1, w_ref[...],
                              preferred_element_type=jnp.float32)

    return pl.pallas_call(
        body,
        in_specs=[
            pl.BlockSpec((2, NPAD, 32), lambda: (0, 0, 0)),
            pl.BlockSpec((1, 32), lambda: (0, 0)),
            pl.BlockSpec((32, 32), lambda: (0, 0)),
        ],
        out_specs=[
            pl.BlockSpec((NPAD, 32), lambda: (0, 0)),
            pl.BlockSpec((NPAD, 32), lambda: (0, 0)),
        ],
        out_shape=[
            jax.ShapeDtypeStruct((NPAD, 32), jnp.float32),
            jax.ShapeDtypeStruct((NPAD, 32), jnp.float32),
        ],
    )(qparts, bias32, ggcWT)


def _tc_gru(mparts, m1, WihT, bih, WhhT, bhh):
    def body(p_ref, x_ref, wi_ref, bi_ref, wh_ref, bh_ref, o_ref):
        agg = p_ref[0] + p_ref[1]
        x = x_ref[...]
        gi = jnp.dot(agg, wi_ref[...],
                     preferred_element_type=jnp.float32) + bi_ref[...]
        gh = jnp.dot(x, wh_ref[...],
                     preferred_element_type=jnp.float32) + bh_ref[...]
        r = jax.nn.sigmoid(gi[:, 0:D] + gh[:, 0:D])
        z = jax.nn.sigmoid(gi[:, D:2 * D] + gh[:, D:2 * D])
        nt = jnp.tanh(gi[:, 2 * D:3 * D] + r * gh[:, 2 * D:3 * D])
        hn = (1.0 - z) * nt + z * x[:, 0:D]
        o_ref[...] = jnp.pad(hn, ((0, 0), (0, 12)))

    return pl.pallas_call(
        body,
        in_specs=[
            pl.BlockSpec((2, NPAD, 32), lambda: (0, 0, 0)),
            pl.BlockSpec((NPAD, 32), lambda: (0, 0)),
            pl.BlockSpec((32, 64), lambda: (0, 0)),
            pl.BlockSpec((1, 64), lambda: (0, 0)),
            pl.BlockSpec((32, 64), lambda: (0, 0)),
            pl.BlockSpec((1, 64), lambda: (0, 0)),
        ],
        out_specs=pl.BlockSpec((NPAD, 32), lambda: (0, 0)),
        out_shape=jax.ShapeDtypeStruct((NPAD, 32), jnp.float32),
    )(mparts, m1, WihT, bih, WhhT, bhh)


def _tc_score(hparts, H, WrelT, brel, WrootT):
    def body(p_ref, h_ref, wr_ref, br_ref, wo_ref, o_ref):
        agg = p_ref[0] + p_ref[1]
        o_ref[...] = (
            jnp.dot(agg, wr_ref[...], preferred_element_type=jnp.float32)
            + br_ref[...]
            + jnp.dot(h_ref[...], wo_ref[...],
                      preferred_element_type=jnp.float32)
        )

    return pl.pallas_call(
        body,
        in_specs=[
            pl.BlockSpec((2, NPAD, 64), lambda: (0, 0, 0)),
            pl.BlockSpec((NPAD, 64), lambda: (0, 0)),
            pl.BlockSpec((64, 8), lambda: (0, 0)),
            pl.BlockSpec((1, 8), lambda: (0, 0)),
            pl.BlockSpec((64, 8), lambda: (0, 0)),
        ],
        out_specs=pl.BlockSpec((NPAD, 8), lambda: (0, 0)),
        out_shape=jax.ShapeDtypeStruct((NPAD, 8), jnp.float32),
    )(hparts, H, WrelT, brel, WrootT)


def _tc_rank(scoreC, scoreR, batchC, batchR):
    BI, BJ = 512, 2048

    def body(si_ref, sj_ref, bi_ref, bj_ref, r_ref, c_ref):
        i0 = pl.program_id(0) * BI
        j0 = pl.program_id(1) * BJ

        @pl.when(pl.program_id(1) == 0)
        def _():
            r_ref[...] = jnp.zeros_like(r_ref)
            c_ref[...] = jnp.zeros_like(c_ref)

        ii = i0 + lax.broadcasted_iota(jnp.int32, (BI, BJ), 0)
        jj = j0 + lax.broadcasted_iota(jnp.int32, (BI, BJ), 1)
        eq = bi_ref[...] == bj_ref[...]
        sj = sj_ref[...]
        si = si_ref[...]
        ahead = (sj > si) | ((sj == si) & (jj < ii))
        contrib = jnp.where(eq & ahead, 1.0, 0.0)
        cgrp = jnp.where(eq, 1.0, 0.0)
        r_ref[...] += jnp.sum(contrib, axis=1, keepdims=True)
        c_ref[...] += jnp.sum(cgrp, axis=1, keepdims=True)

    return pl.pallas_call(
        body,
        grid=(NP2 // BI, NP2 // BJ),
        in_specs=[
            pl.BlockSpec((BI, 1), lambda i, j: (i, 0)),
            pl.BlockSpec((1, BJ), lambda i, j: (0, j)),
            pl.BlockSpec((BI, 1), lambda i, j: (i, 0)),
            pl.BlockSpec((1, BJ), lambda i, j: (0, j)),
        ],
        out_specs=[
            pl.BlockSpec((BI, 1), lambda i, j: (i, 0)),
            pl.BlockSpec((BI, 1), lambda i, j: (i, 0)),
        ],
        out_shape=[
            jax.ShapeDtypeStruct((NP2, 1), jnp.float32),
            jax.ShapeDtypeStruct((NP2, 1), jnp.float32),
        ],
    )(scoreC, scoreR, batchC, batchR)


def _tc_pool(H, score, rank, cnt, batchC):
    B = 1024

    def body(h_ref, s_ref, r_ref, c_ref, b_ref, o_ref):
        @pl.when(pl.program_id(0) == 0)
        def _():
            o_ref[...] = jnp.zeros_like(o_ref)

        kq = jnp.ceil(0.3 * c_ref[...])
        mask = jnp.where(r_ref[...] < kq, 1.0, 0.0)
        hs = h_ref[...] * jnp.tanh(s_ref[...]) * mask
        gid = lax.broadcasted_iota(jnp.int32, (B, 64), 1)
        oh = jnp.where(b_ref[...] == gid, 1.0, 0.0)
        o_ref[...] += lax.dot_general(
            oh, hs, (((0,), (0,)), ((), ())),
            preferred_element_type=jnp.float32)

    return pl.pallas_call(
        body,
        grid=(NP2 // B,),
        in_specs=[
            pl.BlockSpec((B, 64), lambda i: (i, 0)),
            pl.BlockSpec((B, 1), lambda i: (i, 0)),
            pl.BlockSpec((B, 1), lambda i: (i, 0)),
            pl.BlockSpec((B, 1), lambda i: (i, 0)),
            pl.BlockSpec((B, 1), lambda i: (i, 0)),
        ],
        out_specs=pl.BlockSpec((64, 64), lambda i: (0, 0)),
        out_shape=jax.ShapeDtypeStruct((64, 64), jnp.float32),
    )(H, score, rank, cnt, batchC)


def _tc_mlp(g, W1T, b1, W2T, b2, WoT, bo):
    def body(g_ref, w1, b1r, w2, b2r, wo, bor, o_ref):
        a = jnp.dot(g_ref[...], w1[...],
                    preferred_element_type=jnp.float32) + b1r[...]
        a = jnp.where(a >= 0, a, 0.01 * a)
        a = jnp.dot(a, w2[...], preferred_element_type=jnp.float32) + b2r[...]
        a = jnp.where(a >= 0, a, 0.01 * a)
        o_ref[...] = jnp.dot(a, wo[...],
                             preferred_element_type=jnp.float32) + bor[...]

    return pl.pallas_call(
        body,
        in_specs=[
            pl.BlockSpec((64, 64), lambda: (0, 0)),
            pl.BlockSpec((64, 64), lambda: (0, 0)),
            pl.BlockSpec((1, 64), lambda: (0, 0)),
            pl.BlockSpec((64, 32), lambda: (0, 0)),
            pl.BlockSpec((1, 32), lambda: (0, 0)),
            pl.BlockSpec((32, 8), lambda: (0, 0)),
            pl.BlockSpec((1, 8), lambda: (0, 0)),
        ],
        out_specs=pl.BlockSpec((64, 8), lambda: (0, 0)),
        out_shape=jax.ShapeDtypeStruct((64, 8), jnp.float32),
    )(g, W1T, b1, W2T, b2, WoT, bo)


def _padw(a, rows, cols):
    return jnp.pad(a, ((0, rows - a.shape[0]), (0, cols - a.shape[1])))


def kernel(x, edge_index, edge_attr, batch, params):
    p = params
    f = jnp.arange(WF)
    valid = (f < HEADS * D)
    M = ((f[:, None] // D == jnp.arange(16)[None, :]) &
         valid[:, None]).astype(jnp.float32)
    MT = M.T
    R = ((f[:, None] % D == jnp.arange(32)[None, :]) &
         valid[:, None]).astype(jnp.float32)
    attf = jnp.pad(p['gat_att'].reshape(1, HEADS * D), ((0, 0), (0, 8)))

    Wboth = jnp.concatenate(
        [_padw(p['gat_Wl'].T, 32, WF), _padw(p['gat_Wr'].T, 32, WF)], axis=1)
    bboth = jnp.concatenate(
        [jnp.pad(p['gat_bl'], (0, 8)), jnp.pad(p['gat_br'], (0, 8))])
    WeT = jnp.pad(p['gat_We'].T, ((0, 0), (0, 8)))
    bias32 = jnp.pad(p['gat_bias'], (0, 12)).reshape(1, 32)
    ggcWT = _padw(p['ggc_weight'], 32, 32)
    WihT = _padw(p['gru_Wih'].T, 32, 64)
    bih = jnp.pad(p['gru_bih'], (0, 4)).reshape(1, 64)
    WhhT = _padw(p['gru_Whh'].T, 32, 64)
    bhh = jnp.pad(p['gru_bhh'], (0, 4)).reshape(1, 64)
    WrelT = _padw(p['pool_Wrel'].T, 64, 8)
    brel = jnp.pad(p['pool_brel'], (0, 7)).reshape(1, 8)
    WrootT = _padw(p['pool_Wroot'].T, 64, 8)
    W1T = _padw(p['fc1_W'].T, 64, 64)
    b1 = jnp.pad(p['fc1_b'], (0, 24)).reshape(1, 64)
    W2T = _padw(p['fc2_W'].T, 64, 32)
    b2 = jnp.pad(p['fc2_b'], (0, 2)).reshape(1, 32)
    WoT = _padw(p['out_W'].T, 32, 8)
    bo = jnp.pad(p['out_b'], (0, 7)).reshape(1, 8)

    srcP = jnp.concatenate(
        [edge_index[0], jnp.zeros((EPAD - E,), jnp.int32)])
    dstP = jnp.concatenate(
        [edge_index[1], jnp.full((EPAD - E,), N, jnp.int32)])
    src3 = srcP.reshape(NW, KB, BE)
    dst3 = dstP.reshape(NW, KB, BE)
    idxall3 = jnp.concatenate(
        [2 * srcP, 2 * dstP + 1]).reshape(NW, KB2, BE)

    x32 = _padw(x, NPAD, 32)
    eaP = jnp.pad(edge_attr, ((0, EPAD - E), (0, 0)))
    ea_mean = _tc_ea_mean(edge_attr)

    zeros32 = jnp.zeros((2, NPAD, 32), jnp.float32)
    z16 = jnp.zeros((NPAD, 16), jnp.float32)
    z32 = jnp.zeros((NPAD, 32), jnp.float32)

    def gat_layer(h32):
        T2v = _linear(h32, Wboth, bboth, 2504)
        T2 = T2v.reshape(2 * NPAD, WF)
        gAll = _sc_gather_all(T2, idxall3)
        exa = _tc_alpha(gAll, eaP, WeT, attf, M)
        exl = _tc_loop_alpha(T2v, ea_mean, WeT, attf, M, MT)
        denp = _sc_scatter_add(exa, dst3, jnp.stack([exl, z16]))
        den = _tc_den(denp)
        deng = _sc_gather1(den, dst3)
        q = _tc_q(gAll, exa, deng, MT, R)
        qloop = _tc_qloop(T2v, exl, den, MT, R)
        qparts = _sc_scatter_add(q, dst3, jnp.stack([qloop, z32]))
        m1, mW = _tc_m1(qparts, bias32, ggcWT)
        aggm = _sc_gather_scatter_add(mW, src3, dst3, zeros32)
        return _tc_gru(aggm, m1, WihT, bih, WhhT, bhh)

    h1 = gat_layer(x32)
    h2 = gat_layer(h1)

    H = jnp.concatenate([x32[:, :D], h1[:, :D], h2[:, :D]], axis=1)
    H = jnp.pad(H, ((0, 0), (0, 4)))
    hparts = _sc_gather_scatter_add(
        H, src3, dst3, jnp.zeros((2, NPAD, 64), jnp.float32))
    score = _tc_score(hparts, H, WrelT, brel, WrootT)[:, 0:1]

    scoreC = jnp.concatenate(
        [score[:N], jnp.zeros((NP2 - N, 1), jnp.float32)])
    batchC = jnp.concatenate(
        [batch, jnp.full((NP2 - N,), N_GRAPHS, jnp.int32)]).reshape(NP2, 1)
    scoreR = scoreC.reshape(1, NP2)
    batchR = batchC.reshape(1, NP2)
    rank, cnt = _tc_rank(scoreC, scoreR, batchC, batchR)

    H2 = jnp.pad(H[:N], ((0, NP2 - N), (0, 0)))
    g = _tc_pool(H2, scoreC, rank, cnt, batchC)
    out = _tc_mlp(g, W1T, b1, W2T, b2, WoT, bo)
    return out[:, 0]
